# Initial kernel scaffold; baseline (speedup 1.0000x reference)
import jax
import jax.numpy as jnp
from jax.experimental import pallas as pl


def kernel(x, edge_index, edge_attr, W, att_src, att_dst, att_edge, We, bias_gat, W2, b2):
    raise NotImplementedError("write your pallas kernel here")


# speedup vs baseline: 15.2224x; 15.2224x over previous
import functools

import jax
import jax.numpy as jnp
from jax import lax
from jax.experimental import pallas as pl
from jax.experimental.pallas import tpu as pltpu
from jax.experimental.pallas import tpu_sc as plsc

H = 4
C = 256
HC = H * C
NNODE = 10000
NEDGE = 160000
EPAD = 163840
BATCH = 512

TROWS = 9
TFLAT = TROWS * NNODE
TPAD = 90112
TSTRIPE = TPAD // 16

PASS_ROWS = 1280
NPASS = 4
ROWS_PER_TILE = PASS_ROWS // 16
EPB = EPAD // 16
KB = 2048

_mesh = functools.partial(
    plsc.VectorSubcoreMesh, core_axis_name="c", subcore_axis_name="s",
    num_cores=2, num_subcores=16)

_sc_params = pltpu.CompilerParams(needs_layout_passes=False,
                                  use_tc_tiling_on_sc=False)


def _k1_body(x_ref, w_ref, asw_ref, adw_ref, xs_ref, asrc_ref, adst_ref):
    xs = jnp.dot(x_ref[...], w_ref[...], preferred_element_type=jnp.float32)
    xs_ref[...] = xs
    acs = []
    acd = []
    for h in range(H):
        sl = xs[:, h * C:(h + 1) * C]
        acs.append(jnp.sum(sl * asw_ref[h:h + 1, :], axis=1, keepdims=True))
        acd.append(jnp.sum(sl * adw_ref[h:h + 1, :], axis=1, keepdims=True))
    asrc_ref[...] = jnp.concatenate(acs, axis=1)
    adst_ref[...] = jnp.concatenate(acd, axis=1)


def _k1(x, w, att_src, att_dst):
    n, f = x.shape
    bn = 256
    grid = (n + bn - 1) // bn
    return pl.pallas_call(
        _k1_body,
        grid=(grid,),
        in_specs=[
            pl.BlockSpec((bn, f), lambda b: (b, 0)),
            pl.BlockSpec((f, HC), lambda b: (0, 0)),
            pl.BlockSpec((H, C), lambda b: (0, 0)),
            pl.BlockSpec((H, C), lambda b: (0, 0)),
        ],
        out_specs=[
            pl.BlockSpec((bn, HC), lambda b: (b, 0)),
            pl.BlockSpec((bn, H), lambda b: (b, 0)),
            pl.BlockSpec((bn, H), lambda b: (b, 0)),
        ],
        out_shape=[
            jax.ShapeDtypeStruct((n, HC), jnp.float32),
            jax.ShapeDtypeStruct((n, H), jnp.float32),
            jax.ShapeDtypeStruct((n, H), jnp.float32),
        ],
    )(x, w, att_src, att_dst)


def _k1b_body(ea_ref, we_ref, aew_ref, aet_ref):
    cols = []
    for h in range(H):
        cols.append(jnp.sum(we_ref[:, h * C:(h + 1) * C] * aew_ref[h:h + 1, :],
                            axis=1, keepdims=True))
    mt = jnp.concatenate(cols, axis=1)
    aet_ref[...] = lax.dot_general(
        mt, ea_ref[...], (((0,), (1,)), ((), ())),
        preferred_element_type=jnp.float32)


def _k1b(edge_attr, we, att_edge):
    e, ed = edge_attr.shape
    be = 640
    grid = e // be
    return pl.pallas_call(
        _k1b_body,
        grid=(grid,),
        in_specs=[
            pl.BlockSpec((be, ed), lambda b: (b, 0)),
            pl.BlockSpec((ed, HC), lambda b: (0, 0)),
            pl.BlockSpec((H, C), lambda b: (0, 0)),
        ],
        out_specs=pl.BlockSpec((H, be), lambda b: (0, b)),
        out_shape=jax.ShapeDtypeStruct((H, e), jnp.float32),
    )(edge_attr, we, att_edge)


def _k2_body(asrc_hbm, adst_hbm, src_hbm, dst_hbm, aet_hbm,
             expt_hbm, tbl_hbm,
             asrc_v, adst_v, srcb, dstb, aeb, expb, valb,
             idx0, idx1, idx2, idx3, idx4, idx5, idx6, idx7, idx8,
             zerob, tbl_s):
    cid = lax.axis_index("c")
    sid = lax.axis_index("s")
    wid = cid * 16 + sid
    idxs = [idx0, idx1, idx2, idx3, idx4, idx5, idx6, idx7, idx8]

    pltpu.sync_copy(asrc_hbm, asrc_v)
    pltpu.sync_copy(adst_hbm, adst_v)

    z16f = jnp.zeros((16,), jnp.float32)

    def zero_body(i, _):
        zerob[pl.ds(i * 16, 16)] = z16f
        return 0
    lax.fori_loop(0, BATCH // 16, zero_body, 0)
    for i in range(TSTRIPE // BATCH):
        pltpu.sync_copy(zerob, tbl_s.at[pl.ds(sid * TSTRIPE + i * BATCH, BATCH)])
    plsc.subcore_barrier()

    ebase = wid * (EPAD // 32)

    def batch_body(b, _):
        off = ebase + b * BATCH
        pltpu.sync_copy(src_hbm.at[pl.ds(off, BATCH)], srcb)
        pltpu.sync_copy(dst_hbm.at[pl.ds(off, BATCH)], dstb)
        for h in range(H):
            pltpu.sync_copy(aet_hbm.at[h, pl.ds(off, BATCH)],
                            aeb.at[pl.ds(h * BATCH, BATCH)])

        def chunk_body(j, _):
            ds16 = pl.ds(j * 16, 16)
            sv = srcb[ds16]
            dv = dstb[ds16]
            m = sv >= 0
            svc = jnp.maximum(sv, 0)
            mf = jnp.where(m, 1.0, 0.0).astype(jnp.float32)
            valb[ds16] = mf
            for k in range(TROWS):
                idxs[k][ds16] = dv + (k * NNODE)
            for h in range(H):
                g1 = plsc.load_gather(asrc_v, [svc * H + h])
                g2 = plsc.load_gather(adst_v, [dv * H + h])
                av = aeb[pl.ds(h * BATCH + j * 16, 16)]
                l = g1 + g2 + av
                l = jnp.maximum(l, 0.2 * l)
                ex = jnp.exp(l) * mf
                expb[pl.ds(h * BATCH + j * 16, 16)] = ex
                valb[pl.ds((1 + h) * BATCH + j * 16, 16)] = av
                valb[pl.ds((5 + h) * BATCH + j * 16, 16)] = ex
            return 0
        lax.fori_loop(0, BATCH // 16, chunk_body, 0)

        for k in range(TROWS):
            pltpu.sync_copy(valb.at[pl.ds(k * BATCH, BATCH)],
                            tbl_s.at[idxs[k]], add=True)
        for h in range(H):
            pltpu.sync_copy(expb.at[pl.ds(h * BATCH, BATCH)],
                            expt_hbm.at[h, pl.ds(off, BATCH)])
        return 0
    lax.fori_loop(0, EPAD // 32 // BATCH, batch_body, 0)

    plsc.subcore_barrier()
    pltpu.sync_copy(tbl_s.at[pl.ds(sid * TSTRIPE, TSTRIPE)],
                    tbl_hbm.at[cid, pl.ds(sid * TSTRIPE, TSTRIPE)])


def _k2(asrc, adst, srcp, dstp, aetp):
    return pl.kernel(
        _k2_body,
        out_type=[
            jax.ShapeDtypeStruct((H, EPAD), jnp.float32),
            jax.ShapeDtypeStruct((2, TPAD), jnp.float32),
        ],
        mesh=_mesh(),
        compiler_params=_sc_params,
        scratch_types=[
            pltpu.VMEM((NNODE * H,), jnp.float32),
            pltpu.VMEM((NNODE * H,), jnp.float32),
            pltpu.VMEM((BATCH,), jnp.int32),
            pltpu.VMEM((BATCH,), jnp.int32),
            pltpu.VMEM((H * BATCH,), jnp.float32),
            pltpu.VMEM((H * BATCH,), jnp.float32),
            pltpu.VMEM((TROWS * BATCH,), jnp.float32),
        ] + [pltpu.VMEM((BATCH,), jnp.int32) for _ in range(TROWS)]
        + [
            pltpu.VMEM((BATCH,), jnp.float32),
            pltpu.VMEM_SHARED((TPAD,), jnp.float32),
        ],
    )(asrc, adst, srcp, dstp, aetp)


def _k5_body(xs_hbm, src_hbm, dst_hbm, expt_hbm, msg_hbm,
             srcb, dstb, expb, eidc, locr, gbufa, gbufb, acc_s,
             sema, semb):
    cid = lax.axis_index("c")
    sid = lax.axis_index("s")
    ebase = sid * EPB

    z16f = jnp.zeros((16,), jnp.float32)
    lane = jnp.arange(16, dtype=jnp.int32)

    def pass_body(p, _):
        lo = cid * (NPASS * PASS_ROWS) + p * PASS_ROWS
        hi = lo + PASS_ROWS

        for i in range(16):
            def zb_body(q, _):
                gbufa[i, pl.ds(q * 16, 16)] = z16f
                return 0
            lax.fori_loop(0, HC // 16, zb_body, 0)
        for k in range(ROWS_PER_TILE // 16):
            pltpu.sync_copy(
                gbufa, acc_s.at[pl.ds(sid * ROWS_PER_TILE + k * 16, 16), :])
        plsc.subcore_barrier()

        def seg_body(t, _):
            off = ebase + t * KB
            pltpu.sync_copy(src_hbm.at[pl.ds(off, KB)], srcb)
            pltpu.sync_copy(dst_hbm.at[pl.ds(off, KB)], dstb)
            for h in range(H):
                pltpu.sync_copy(expt_hbm.at[h, pl.ds(off, KB)],
                                expb.at[pl.ds(h * KB, KB)])

            def cmp_body(j, cnt):
                ds16 = pl.ds(j * 16, 16)
                sv = srcb[ds16]
                dv = dstb[ds16]
                m = (dv >= lo) & (dv < hi) & (sv >= 0)
                plsc.store_compressed(eidc.at[pl.ds(cnt, 16)],
                                      lane + j * 16, mask=m)
                return cnt + jnp.sum(m.astype(jnp.int32))
            cnt = lax.fori_loop(0, KB // 16, cmp_body, 0)

            z16i = jnp.zeros((16,), jnp.int32)
            eidc[pl.ds(cnt, 16)] = z16i
            eidc[pl.ds(cnt + 16, 16)] = z16i
            ntrip = (cnt + 15) // 16
            ngrp = ntrip + (ntrip & 1)

            def prep(g, buf, sem):
                eidv = eidc[pl.ds(g * 16, 16)]
                srcv = jnp.maximum(plsc.load_gather(srcb, [eidv]), 0)
                pltpu.async_copy(xs_hbm.at[srcv], buf, sem)

            def consume(g, buf, sem):
                eidv = eidc[pl.ds(g * 16, 16)]
                validv = (g * 16 + lane) < cnt
                vmf = jnp.where(validv, 1.0, 0.0).astype(jnp.float32)
                srcv = jnp.maximum(plsc.load_gather(srcb, [eidv]), 0)
                pltpu.make_async_copy(xs_hbm.at[srcv], buf, sem).wait()
                locv = jnp.where(validv,
                                 plsc.load_gather(dstb, [eidv]) - lo, -1)
                locr[pl.ds(0, 16)] = locv
                for h in range(H):
                    expv = plsc.load_gather(expb, [eidv + h * KB]) * vmf
                    for e in range(16):
                        av = z16f + expv[e]
                        for q in range(C // 16):
                            sl = pl.ds(h * C + q * 16, 16)
                            buf[e, sl] = buf[e, sl] * av
                pltpu.sync_copy(
                    buf, acc_s.at[plsc.Indices(locr, ignored_value=-1)],
                    add=True)

            @pl.when(cnt > 0)
            def _():
                prep(0, gbufa, sema)
                prep(1, gbufb, semb)

                def pair_body(i, _):
                    g = 2 * i
                    consume(g, gbufa, sema)
                    prep(g + 2, gbufa, sema)
                    consume(g + 1, gbufb, semb)
                    prep(g + 3, gbufb, semb)
                    return 0
                lax.fori_loop(0, ngrp // 2 - 1, pair_body, 0)
                consume(ngrp - 2, gbufa, sema)
                consume(ngrp - 1, gbufb, semb)
            return 0
        lax.fori_loop(0, EPB // KB, seg_body, 0)

        plsc.subcore_barrier()
        row0 = lo + sid * ROWS_PER_TILE

        @pl.when(row0 < NNODE)
        def _():
            pltpu.sync_copy(
                acc_s.at[pl.ds(sid * ROWS_PER_TILE, ROWS_PER_TILE), :],
                msg_hbm.at[pl.ds(row0, ROWS_PER_TILE), :])
        return 0
    lax.fori_loop(0, NPASS, pass_body, 0)


def _k5(xs, srcp, dstp, expt):
    return pl.kernel(
        _k5_body,
        out_type=jax.ShapeDtypeStruct((NNODE, HC), jnp.float32),
        mesh=_mesh(),
        compiler_params=_sc_params,
        scratch_types=[
            pltpu.VMEM((KB,), jnp.int32),
            pltpu.VMEM((KB,), jnp.int32),
            pltpu.VMEM((H * KB,), jnp.float32),
            pltpu.VMEM((KB + 64,), jnp.int32),
            pltpu.VMEM((16,), jnp.int32),
            pltpu.VMEM((16, HC), jnp.float32),
            pltpu.VMEM((16, HC), jnp.float32),
            pltpu.VMEM_SHARED((PASS_ROWS, HC), jnp.float32),
            pltpu.SemaphoreType.DMA,
            pltpu.SemaphoreType.DMA,
        ],
    )(xs, srcp, dstp, expt)


def _k6_body(msg_ref, xs_ref, asrc_ref, adst_ref, ta_ref, tb_ref,
             bias_ref, w2_ref, b2_ref, out_ref):
    deg = ta_ref[:, 0:1] + tb_ref[:, 0:1]
    degc = jnp.maximum(deg, 1.0)
    gs = []
    for h in range(H):
        aes = ta_ref[:, 1 + h:2 + h] + tb_ref[:, 1 + h:2 + h]
        dnp = ta_ref[:, 5 + h:6 + h] + tb_ref[:, 5 + h:6 + h]
        ael = aes / degc
        ll = asrc_ref[:, h:h + 1] + adst_ref[:, h:h + 1] + ael
        ll = jnp.maximum(ll, 0.2 * ll)
        exl = jnp.exp(ll)
        den = dnp + exl + 1e-16
        gh = (msg_ref[:, h * C:(h + 1) * C]
              + exl * xs_ref[:, h * C:(h + 1) * C]) / den
        gs.append(gh)
    gat = jnp.concatenate(gs, axis=1) + bias_ref[...]
    out_ref[...] = (jnp.dot(gat, w2_ref[...], preferred_element_type=jnp.float32)
                    + b2_ref[...])


def _k6(msg, xs, asrc, adst, ta, tb, bias, w2, b2):
    n = msg.shape[0]
    f = w2.shape[1]
    bn = 256
    grid = (n + bn - 1) // bn
    return pl.pallas_call(
        _k6_body,
        grid=(grid,),
        in_specs=[
            pl.BlockSpec((bn, HC), lambda b: (b, 0)),
            pl.BlockSpec((bn, HC), lambda b: (b, 0)),
            pl.BlockSpec((bn, H), lambda b: (b, 0)),
            pl.BlockSpec((bn, H), lambda b: (b, 0)),
            pl.BlockSpec((bn, TROWS), lambda b: (b, 0)),
            pl.BlockSpec((bn, TROWS), lambda b: (b, 0)),
            pl.BlockSpec((1, HC), lambda b: (0, 0)),
            pl.BlockSpec((HC, f), lambda b: (0, 0)),
            pl.BlockSpec((1, f), lambda b: (0, 0)),
        ],
        out_specs=pl.BlockSpec((bn, f), lambda b: (b, 0)),
        out_shape=jax.ShapeDtypeStruct((n, f), jnp.float32),
    )(msg, xs, asrc, adst, ta, tb, bias, w2, b2)


def kernel(x, edge_index, edge_attr, W, att_src, att_dst, att_edge,
           We, bias_gat, W2, b2):
    src = edge_index[0].astype(jnp.int32)
    dst = edge_index[1].astype(jnp.int32)
    npad = EPAD - src.shape[0]
    srcp = jnp.concatenate([src, jnp.full((npad,), -1, jnp.int32)])
    dstp = jnp.concatenate([dst, jnp.zeros((npad,), jnp.int32)])

    xs, asrc, adst = _k1(x, W, att_src, att_dst)
    aet = _k1b(edge_attr, We, att_edge)
    aetp = jnp.concatenate([aet, jnp.zeros((H, npad), jnp.float32)], axis=1)

    expt, tbl = _k2(asrc.reshape(-1), adst.reshape(-1), srcp, dstp, aetp)
    msg = _k5(xs, srcp, dstp, expt)

    ta = tbl[0, :TFLAT].reshape(TROWS, NNODE).T
    tb = tbl[1, :TFLAT].reshape(TROWS, NNODE).T
    return _k6(msg, xs, asrc, adst, ta, tb,
               bias_gat.reshape(1, HC), W2, b2.reshape(1, -1))


# speedup vs baseline: 15.6079x; 1.0253x over previous
import functools

import jax
import jax.numpy as jnp
from jax import lax
from jax.experimental import pallas as pl
from jax.experimental.pallas import tpu as pltpu
from jax.experimental.pallas import tpu_sc as plsc

H = 4
C = 256
HC = H * C
NNODE = 10000
NEDGE = 160000
EPAD = 163840
BATCH = 512

TROWS = 9
TFLAT = TROWS * NNODE
TPAD = 90112
TSTRIPE = TPAD // 16

PASS_ROWS = 1024
NPASS = 5
MSGPAD = 2 * NPASS * PASS_ROWS
ROWS_PER_TILE = PASS_ROWS // 16
EPB = EPAD // 16
KB = 2048

_mesh = functools.partial(
    plsc.VectorSubcoreMesh, core_axis_name="c", subcore_axis_name="s",
    num_cores=2, num_subcores=16)

_sc_params = pltpu.CompilerParams(needs_layout_passes=False,
                                  use_tc_tiling_on_sc=False)


def _k1_body(x_ref, w_ref, asw_ref, adw_ref, xs_ref, asrc_ref, adst_ref):
    xs = jnp.dot(x_ref[...], w_ref[...], preferred_element_type=jnp.float32)
    xs_ref[...] = xs
    acs = []
    acd = []
    for h in range(H):
        sl = xs[:, h * C:(h + 1) * C]
        acs.append(jnp.sum(sl * asw_ref[h:h + 1, :], axis=1, keepdims=True))
        acd.append(jnp.sum(sl * adw_ref[h:h + 1, :], axis=1, keepdims=True))
    asrc_ref[...] = jnp.concatenate(acs, axis=1)
    adst_ref[...] = jnp.concatenate(acd, axis=1)


def _k1(x, w, att_src, att_dst):
    n, f = x.shape
    bn = 256
    grid = (n + bn - 1) // bn
    return pl.pallas_call(
        _k1_body,
        grid=(grid,),
        in_specs=[
            pl.BlockSpec((bn, f), lambda b: (b, 0)),
            pl.BlockSpec((f, HC), lambda b: (0, 0)),
            pl.BlockSpec((H, C), lambda b: (0, 0)),
            pl.BlockSpec((H, C), lambda b: (0, 0)),
        ],
        out_specs=[
            pl.BlockSpec((bn, HC), lambda b: (b, 0)),
            pl.BlockSpec((bn, H), lambda b: (b, 0)),
            pl.BlockSpec((bn, H), lambda b: (b, 0)),
        ],
        out_shape=[
            jax.ShapeDtypeStruct((n, HC), jnp.float32),
            jax.ShapeDtypeStruct((n, H), jnp.float32),
            jax.ShapeDtypeStruct((n, H), jnp.float32),
        ],
    )(x, w, att_src, att_dst)


def _k1b_body(ea_ref, we_ref, aew_ref, aet_ref):
    cols = []
    for h in range(H):
        cols.append(jnp.sum(we_ref[:, h * C:(h + 1) * C] * aew_ref[h:h + 1, :],
                            axis=1, keepdims=True))
    mt = jnp.concatenate(cols, axis=1)
    aet_ref[...] = lax.dot_general(
        mt, ea_ref[...], (((0,), (1,)), ((), ())),
        preferred_element_type=jnp.float32)


def _k1b(edge_attr, we, att_edge):
    e, ed = edge_attr.shape
    be = 640
    grid = e // be
    return pl.pallas_call(
        _k1b_body,
        grid=(grid,),
        in_specs=[
            pl.BlockSpec((be, ed), lambda b: (b, 0)),
            pl.BlockSpec((ed, HC), lambda b: (0, 0)),
            pl.BlockSpec((H, C), lambda b: (0, 0)),
        ],
        out_specs=pl.BlockSpec((H, be), lambda b: (0, b)),
        out_shape=jax.ShapeDtypeStruct((H, e), jnp.float32),
    )(edge_attr, we, att_edge)


def _k2_body(asrc_hbm, adst_hbm, src_hbm, dst_hbm, aet_hbm,
             expt_hbm, tbl_hbm,
             asrc_v, adst_v, srcb, dstb, aeb, expb, valb,
             idx0, idx1, idx2, idx3, idx4, idx5, idx6, idx7, idx8,
             zerob, tbl_s):
    cid = lax.axis_index("c")
    sid = lax.axis_index("s")
    wid = cid * 16 + sid
    idxs = [idx0, idx1, idx2, idx3, idx4, idx5, idx6, idx7, idx8]

    pltpu.sync_copy(asrc_hbm, asrc_v)
    pltpu.sync_copy(adst_hbm, adst_v)

    z16f = jnp.zeros((16,), jnp.float32)

    def zero_body(i, _):
        zerob[pl.ds(i * 16, 16)] = z16f
        return 0
    lax.fori_loop(0, BATCH // 16, zero_body, 0)
    for i in range(TSTRIPE // BATCH):
        pltpu.sync_copy(zerob, tbl_s.at[pl.ds(sid * TSTRIPE + i * BATCH, BATCH)])
    plsc.subcore_barrier()

    ebase = wid * (EPAD // 32)

    def batch_body(b, _):
        off = ebase + b * BATCH
        pltpu.sync_copy(src_hbm.at[pl.ds(off, BATCH)], srcb)
        pltpu.sync_copy(dst_hbm.at[pl.ds(off, BATCH)], dstb)
        for h in range(H):
            pltpu.sync_copy(aet_hbm.at[h, pl.ds(off, BATCH)],
                            aeb.at[pl.ds(h * BATCH, BATCH)])

        def chunk_body(j, _):
            ds16 = pl.ds(j * 16, 16)
            sv = srcb[ds16]
            dv = dstb[ds16]
            m = sv >= 0
            svc = jnp.maximum(sv, 0)
            mf = jnp.where(m, 1.0, 0.0).astype(jnp.float32)
            valb[ds16] = mf
            for k in range(TROWS):
                idxs[k][ds16] = dv + (k * NNODE)
            for h in range(H):
                g1 = plsc.load_gather(asrc_v, [svc * H + h])
                g2 = plsc.load_gather(adst_v, [dv * H + h])
                av = aeb[pl.ds(h * BATCH + j * 16, 16)]
                l = g1 + g2 + av
                l = jnp.maximum(l, 0.2 * l)
                ex = jnp.exp(l) * mf
                expb[pl.ds(h * BATCH + j * 16, 16)] = ex
                valb[pl.ds((1 + h) * BATCH + j * 16, 16)] = av
                valb[pl.ds((5 + h) * BATCH + j * 16, 16)] = ex
            return 0
        lax.fori_loop(0, BATCH // 16, chunk_body, 0)

        for k in range(TROWS):
            pltpu.sync_copy(valb.at[pl.ds(k * BATCH, BATCH)],
                            tbl_s.at[idxs[k]], add=True)
        for h in range(H):
            pltpu.sync_copy(expb.at[pl.ds(h * BATCH, BATCH)],
                            expt_hbm.at[h, pl.ds(off, BATCH)])
        return 0
    lax.fori_loop(0, EPAD // 32 // BATCH, batch_body, 0)

    plsc.subcore_barrier()
    pltpu.sync_copy(tbl_s.at[pl.ds(sid * TSTRIPE, TSTRIPE)],
                    tbl_hbm.at[cid, pl.ds(sid * TSTRIPE, TSTRIPE)])


def _k2(asrc, adst, srcp, dstp, aetp):
    return pl.kernel(
        _k2_body,
        out_type=[
            jax.ShapeDtypeStruct((H, EPAD), jnp.float32),
            jax.ShapeDtypeStruct((2, TPAD), jnp.float32),
        ],
        mesh=_mesh(),
        compiler_params=_sc_params,
        scratch_types=[
            pltpu.VMEM((NNODE * H,), jnp.float32),
            pltpu.VMEM((NNODE * H,), jnp.float32),
            pltpu.VMEM((BATCH,), jnp.int32),
            pltpu.VMEM((BATCH,), jnp.int32),
            pltpu.VMEM((H * BATCH,), jnp.float32),
            pltpu.VMEM((H * BATCH,), jnp.float32),
            pltpu.VMEM((TROWS * BATCH,), jnp.float32),
        ] + [pltpu.VMEM((BATCH,), jnp.int32) for _ in range(TROWS)]
        + [
            pltpu.VMEM((BATCH,), jnp.float32),
            pltpu.VMEM_SHARED((TPAD,), jnp.float32),
        ],
    )(asrc, adst, srcp, dstp, aetp)


def _k5_body(xs_hbm, src_hbm, dst_hbm, expt_hbm, msg_hbm,
             srcb, dstb, expb, eidc,
             locr0, locr1, locr2, gbuf0, gbuf1, gbuf2, acc_s,
             sg0, sg1, sg2, ss0, ss1, ss2):
    cid = lax.axis_index("c")
    sid = lax.axis_index("s")
    ebase = sid * EPB
    bufs = [gbuf0, gbuf1, gbuf2]
    locrs = [locr0, locr1, locr2]
    gsems = [sg0, sg1, sg2]
    ssems = [ss0, ss1, ss2]

    z16f = jnp.zeros((16,), jnp.float32)
    lane = jnp.arange(16, dtype=jnp.int32)

    def pass_body(p, _):
        lo = cid * (NPASS * PASS_ROWS) + p * PASS_ROWS
        hi = lo + PASS_ROWS

        for i in range(16):
            def zb_body(q, _):
                gbuf0[i, pl.ds(q * 16, 16)] = z16f
                return 0
            lax.fori_loop(0, HC // 16, zb_body, 0)
        for k in range(ROWS_PER_TILE // 16):
            pltpu.sync_copy(
                gbuf0, acc_s.at[pl.ds(sid * ROWS_PER_TILE + k * 16, 16), :])
        plsc.subcore_barrier()

        def seg_body(t, _):
            off = ebase + t * KB
            pltpu.sync_copy(src_hbm.at[pl.ds(off, KB)], srcb)
            pltpu.sync_copy(dst_hbm.at[pl.ds(off, KB)], dstb)
            for h in range(H):
                pltpu.sync_copy(expt_hbm.at[h, pl.ds(off, KB)],
                                expb.at[pl.ds(h * KB, KB)])

            def cmp_body(j, cnt):
                ds16 = pl.ds(j * 16, 16)
                sv = srcb[ds16]
                dv = dstb[ds16]
                m = (dv >= lo) & (dv < hi) & (sv >= 0)
                plsc.store_compressed(eidc.at[pl.ds(cnt, 16)],
                                      lane + j * 16, mask=m)
                return cnt + jnp.sum(m.astype(jnp.int32))
            cnt = lax.fori_loop(0, KB // 16, cmp_body, 0)

            z16i = jnp.zeros((16,), jnp.int32)
            eidc[pl.ds(cnt, 16)] = z16i
            eidc[pl.ds(cnt + 16, 16)] = z16i
            eidc[pl.ds(cnt + 32, 16)] = z16i
            ntrip = (cnt + 15) // 16
            ngrp = ((ntrip + 2) // 3) * 3

            def prep(g, b):
                eidv = eidc[pl.ds(g * 16, 16)]
                srcv = jnp.maximum(plsc.load_gather(srcb, [eidv]), 0)
                pltpu.async_copy(xs_hbm.at[srcv], bufs[b], gsems[b])

            def swait(b):
                pltpu.make_async_copy(
                    bufs[b],
                    acc_s.at[plsc.Indices(locrs[b], ignored_value=-1)],
                    ssems[b]).wait()

            @pl.when(cnt > 0)
            def _():
                prep(0, 0)
                prep(1, 1)

                def tri_body(i, _):
                    for b in range(3):
                        g = 3 * i + b
                        eidv = eidc[pl.ds(g * 16, 16)]
                        validv = (g * 16 + lane) < cnt
                        vmf = jnp.where(validv, 1.0, 0.0).astype(jnp.float32)
                        srcv = jnp.maximum(
                            plsc.load_gather(srcb, [eidv]), 0)
                        pltpu.make_async_copy(
                            xs_hbm.at[srcv], bufs[b], gsems[b]).wait()
                        locv = jnp.where(
                            validv, plsc.load_gather(dstb, [eidv]) - lo, -1)
                        locrs[b][pl.ds(0, 16)] = locv
                        buf = bufs[b]
                        for h in range(H):
                            expv = plsc.load_gather(
                                expb, [eidv + h * KB]) * vmf
                            for e in range(16):
                                av = z16f + expv[e]
                                for q in range(C // 16):
                                    sl = pl.ds(h * C + q * 16, 16)
                                    buf[e, sl] = buf[e, sl] * av
                        pltpu.async_copy(
                            buf,
                            acc_s.at[plsc.Indices(locrs[b],
                                                  ignored_value=-1)],
                            ssems[b], add=True)
                        b2 = (b + 2) % 3

                        @pl.when(g >= 1)
                        def _():
                            swait(b2)

                        @pl.when(g + 2 < ngrp)
                        def _():
                            prep(g + 2, b2)
                    return 0
                lax.fori_loop(0, ngrp // 3, tri_body, 0)
                swait(2)
            return 0
        lax.fori_loop(0, EPB // KB, seg_body, 0)

        plsc.subcore_barrier()
        row0 = lo + sid * ROWS_PER_TILE
        pltpu.sync_copy(
            acc_s.at[pl.ds(sid * ROWS_PER_TILE, ROWS_PER_TILE), :],
            msg_hbm.at[pl.ds(row0, ROWS_PER_TILE), :])
        return 0
    lax.fori_loop(0, NPASS, pass_body, 0)


def _k5(xs, srcp, dstp, expt):
    return pl.kernel(
        _k5_body,
        out_type=jax.ShapeDtypeStruct((MSGPAD, HC), jnp.float32),
        mesh=_mesh(),
        compiler_params=_sc_params,
        scratch_types=[
            pltpu.VMEM((KB,), jnp.int32),
            pltpu.VMEM((KB,), jnp.int32),
            pltpu.VMEM((H * KB,), jnp.float32),
            pltpu.VMEM((KB + 64,), jnp.int32),
            pltpu.VMEM((16,), jnp.int32),
            pltpu.VMEM((16,), jnp.int32),
            pltpu.VMEM((16,), jnp.int32),
            pltpu.VMEM((16, HC), jnp.float32),
            pltpu.VMEM((16, HC), jnp.float32),
            pltpu.VMEM((16, HC), jnp.float32),
            pltpu.VMEM_SHARED((PASS_ROWS, HC), jnp.float32),
            pltpu.SemaphoreType.DMA,
            pltpu.SemaphoreType.DMA,
            pltpu.SemaphoreType.DMA,
            pltpu.SemaphoreType.DMA,
            pltpu.SemaphoreType.DMA,
            pltpu.SemaphoreType.DMA,
        ],
    )(xs, srcp, dstp, expt)


def _k6_body(msg_ref, xs_ref, asrc_ref, adst_ref, ta_ref, tb_ref,
             bias_ref, w2_ref, b2_ref, out_ref):
    deg = ta_ref[:, 0:1] + tb_ref[:, 0:1]
    degc = jnp.maximum(deg, 1.0)
    gs = []
    for h in range(H):
        aes = ta_ref[:, 1 + h:2 + h] + tb_ref[:, 1 + h:2 + h]
        dnp = ta_ref[:, 5 + h:6 + h] + tb_ref[:, 5 + h:6 + h]
        ael = aes / degc
        ll = asrc_ref[:, h:h + 1] + adst_ref[:, h:h + 1] + ael
        ll = jnp.maximum(ll, 0.2 * ll)
        exl = jnp.exp(ll)
        den = dnp + exl + 1e-16
        gh = (msg_ref[:, h * C:(h + 1) * C]
              + exl * xs_ref[:, h * C:(h + 1) * C]) / den
        gs.append(gh)
    gat = jnp.concatenate(gs, axis=1) + bias_ref[...]
    out_ref[...] = (jnp.dot(gat, w2_ref[...], preferred_element_type=jnp.float32)
                    + b2_ref[...])


def _k6(msg, xs, asrc, adst, ta, tb, bias, w2, b2):
    n = msg.shape[0]
    f = w2.shape[1]
    bn = 256
    grid = (n + bn - 1) // bn
    return pl.pallas_call(
        _k6_body,
        grid=(grid,),
        in_specs=[
            pl.BlockSpec((bn, HC), lambda b: (b, 0)),
            pl.BlockSpec((bn, HC), lambda b: (b, 0)),
            pl.BlockSpec((bn, H), lambda b: (b, 0)),
            pl.BlockSpec((bn, H), lambda b: (b, 0)),
            pl.BlockSpec((bn, TROWS), lambda b: (b, 0)),
            pl.BlockSpec((bn, TROWS), lambda b: (b, 0)),
            pl.BlockSpec((1, HC), lambda b: (0, 0)),
            pl.BlockSpec((HC, f), lambda b: (0, 0)),
            pl.BlockSpec((1, f), lambda b: (0, 0)),
        ],
        out_specs=pl.BlockSpec((bn, f), lambda b: (b, 0)),
        out_shape=jax.ShapeDtypeStruct((n, f), jnp.float32),
    )(msg, xs, asrc, adst, ta, tb, bias, w2, b2)


def kernel(x, edge_index, edge_attr, W, att_src, att_dst, att_edge,
           We, bias_gat, W2, b2):
    src = edge_index[0].astype(jnp.int32)
    dst = edge_index[1].astype(jnp.int32)
    npad = EPAD - src.shape[0]
    srcp = jnp.concatenate([src, jnp.full((npad,), -1, jnp.int32)])
    dstp = jnp.concatenate([dst, jnp.zeros((npad,), jnp.int32)])

    xs, asrc, adst = _k1(x, W, att_src, att_dst)
    aet = _k1b(edge_attr, We, att_edge)
    aetp = jnp.concatenate([aet, jnp.zeros((H, npad), jnp.float32)], axis=1)

    expt, tbl = _k2(asrc.reshape(-1), adst.reshape(-1), srcp, dstp, aetp)
    msg = _k5(xs, srcp, dstp, expt)[:NNODE]

    ta = tbl[0, :TFLAT].reshape(TROWS, NNODE).T
    tb = tbl[1, :TFLAT].reshape(TROWS, NNODE).T
    return _k6(msg, xs, asrc, adst, ta, tb,
               bias_gat.reshape(1, HC), W2, b2.reshape(1, -1))


# speedup vs baseline: 16.3739x; 1.0491x over previous
import functools

import jax
import jax.numpy as jnp
from jax import lax
from jax.experimental import pallas as pl
from jax.experimental.pallas import tpu as pltpu
from jax.experimental.pallas import tpu_sc as plsc

H = 4
C = 256
HC = H * C
NNODE = 10000
NEDGE = 160000
EPAD = 163840
BATCH = 512

TROWS = 9
TFLAT = TROWS * NNODE
TPAD = 90112
TSTRIPE = TPAD // 16

PASS_ROWS = 1024
NPASS = 5
MSGPAD = 2 * NPASS * PASS_ROWS
ROWS_PER_TILE = PASS_ROWS // 16
EPB = EPAD // 16
KB = 2048

_mesh = functools.partial(
    plsc.VectorSubcoreMesh, core_axis_name="c", subcore_axis_name="s",
    num_cores=2, num_subcores=16)

_sc_params = pltpu.CompilerParams(needs_layout_passes=False,
                                  use_tc_tiling_on_sc=False)


def _k1_body(x_ref, w_ref, asw_ref, adw_ref, xs_ref, asrc_ref, adst_ref):
    xs = jnp.dot(x_ref[...], w_ref[...], preferred_element_type=jnp.float32)
    xs_ref[...] = xs
    acs = []
    acd = []
    for h in range(H):
        sl = xs[:, h * C:(h + 1) * C]
        acs.append(jnp.sum(sl * asw_ref[h:h + 1, :], axis=1, keepdims=True))
        acd.append(jnp.sum(sl * adw_ref[h:h + 1, :], axis=1, keepdims=True))
    asrc_ref[...] = jnp.concatenate(acs, axis=1)
    adst_ref[...] = jnp.concatenate(acd, axis=1)


def _k1(x, w, att_src, att_dst):
    n, f = x.shape
    bn = 256
    grid = (n + bn - 1) // bn
    return pl.pallas_call(
        _k1_body,
        grid=(grid,),
        in_specs=[
            pl.BlockSpec((bn, f), lambda b: (b, 0)),
            pl.BlockSpec((f, HC), lambda b: (0, 0)),
            pl.BlockSpec((H, C), lambda b: (0, 0)),
            pl.BlockSpec((H, C), lambda b: (0, 0)),
        ],
        out_specs=[
            pl.BlockSpec((bn, HC), lambda b: (b, 0)),
            pl.BlockSpec((bn, H), lambda b: (b, 0)),
            pl.BlockSpec((bn, H), lambda b: (b, 0)),
        ],
        out_shape=[
            jax.ShapeDtypeStruct((n, HC), jnp.float32),
            jax.ShapeDtypeStruct((n, H), jnp.float32),
            jax.ShapeDtypeStruct((n, H), jnp.float32),
        ],
    )(x, w, att_src, att_dst)


def _k1b_body(ea_ref, we_ref, aew_ref, aet_ref):
    cols = []
    for h in range(H):
        cols.append(jnp.sum(we_ref[:, h * C:(h + 1) * C] * aew_ref[h:h + 1, :],
                            axis=1, keepdims=True))
    mt = jnp.concatenate(cols, axis=1)
    aet_ref[...] = lax.dot_general(
        mt, ea_ref[...], (((0,), (1,)), ((), ())),
        preferred_element_type=jnp.float32)


def _k1b(edge_attr, we, att_edge):
    e, ed = edge_attr.shape
    be = 640
    grid = e // be
    return pl.pallas_call(
        _k1b_body,
        grid=(grid,),
        in_specs=[
            pl.BlockSpec((be, ed), lambda b: (b, 0)),
            pl.BlockSpec((ed, HC), lambda b: (0, 0)),
            pl.BlockSpec((H, C), lambda b: (0, 0)),
        ],
        out_specs=pl.BlockSpec((H, be), lambda b: (0, b)),
        out_shape=jax.ShapeDtypeStruct((H, e), jnp.float32),
    )(edge_attr, we, att_edge)


def _k2_body(asrc_hbm, adst_hbm, src_hbm, dst_hbm, aet_hbm,
             expt_hbm, tbl_hbm,
             asrc_v, adst_v, srcb, dstb, aeb, expb, valb,
             idx0, idx1, idx2, idx3, idx4, idx5, idx6, idx7, idx8,
             zerob, tbl_s):
    cid = lax.axis_index("c")
    sid = lax.axis_index("s")
    wid = cid * 16 + sid
    idxs = [idx0, idx1, idx2, idx3, idx4, idx5, idx6, idx7, idx8]

    pltpu.sync_copy(asrc_hbm, asrc_v)
    pltpu.sync_copy(adst_hbm, adst_v)

    z16f = jnp.zeros((16,), jnp.float32)

    def zero_body(i, _):
        zerob[pl.ds(i * 16, 16)] = z16f
        return 0
    lax.fori_loop(0, BATCH // 16, zero_body, 0)
    for i in range(TSTRIPE // BATCH):
        pltpu.sync_copy(zerob, tbl_s.at[pl.ds(sid * TSTRIPE + i * BATCH, BATCH)])
    plsc.subcore_barrier()

    ebase = wid * (EPAD // 32)

    def batch_body(b, _):
        off = ebase + b * BATCH
        pltpu.sync_copy(src_hbm.at[pl.ds(off, BATCH)], srcb)
        pltpu.sync_copy(dst_hbm.at[pl.ds(off, BATCH)], dstb)
        for h in range(H):
            pltpu.sync_copy(aet_hbm.at[h, pl.ds(off, BATCH)],
                            aeb.at[pl.ds(h * BATCH, BATCH)])

        def chunk_body(j, _):
            ds16 = pl.ds(j * 16, 16)
            sv = srcb[ds16]
            dv = dstb[ds16]
            m = sv >= 0
            svc = jnp.maximum(sv, 0)
            mf = jnp.where(m, 1.0, 0.0).astype(jnp.float32)
            valb[ds16] = mf
            for k in range(TROWS):
                idxs[k][ds16] = dv + (k * NNODE)
            for h in range(H):
                g1 = plsc.load_gather(asrc_v, [svc * H + h])
                g2 = plsc.load_gather(adst_v, [dv * H + h])
                av = aeb[pl.ds(h * BATCH + j * 16, 16)]
                l = g1 + g2 + av
                l = jnp.maximum(l, 0.2 * l)
                ex = jnp.exp(l) * mf
                expb[pl.ds(h * BATCH + j * 16, 16)] = ex
                valb[pl.ds((1 + h) * BATCH + j * 16, 16)] = av
                valb[pl.ds((5 + h) * BATCH + j * 16, 16)] = ex
            return 0
        lax.fori_loop(0, BATCH // 16, chunk_body, 0)

        for k in range(TROWS):
            pltpu.sync_copy(valb.at[pl.ds(k * BATCH, BATCH)],
                            tbl_s.at[idxs[k]], add=True)
        for h in range(H):
            pltpu.sync_copy(expb.at[pl.ds(h * BATCH, BATCH)],
                            expt_hbm.at[h, pl.ds(off, BATCH)])
        return 0
    lax.fori_loop(0, EPAD // 32 // BATCH, batch_body, 0)

    plsc.subcore_barrier()
    pltpu.sync_copy(tbl_s.at[pl.ds(sid * TSTRIPE, TSTRIPE)],
                    tbl_hbm.at[cid, pl.ds(sid * TSTRIPE, TSTRIPE)])


def _k2(asrc, adst, srcp, dstp, aetp):
    return pl.kernel(
        _k2_body,
        out_type=[
            jax.ShapeDtypeStruct((H, EPAD), jnp.float32),
            jax.ShapeDtypeStruct((2, TPAD), jnp.float32),
        ],
        mesh=_mesh(),
        compiler_params=_sc_params,
        scratch_types=[
            pltpu.VMEM((NNODE * H,), jnp.float32),
            pltpu.VMEM((NNODE * H,), jnp.float32),
            pltpu.VMEM((BATCH,), jnp.int32),
            pltpu.VMEM((BATCH,), jnp.int32),
            pltpu.VMEM((H * BATCH,), jnp.float32),
            pltpu.VMEM((H * BATCH,), jnp.float32),
            pltpu.VMEM((TROWS * BATCH,), jnp.float32),
        ] + [pltpu.VMEM((BATCH,), jnp.int32) for _ in range(TROWS)]
        + [
            pltpu.VMEM((BATCH,), jnp.float32),
            pltpu.VMEM_SHARED((TPAD,), jnp.float32),
        ],
    )(asrc, adst, srcp, dstp, aetp)


def _k5_body(xs_hbm, src_hbm, dst_hbm, expt_hbm, msg_hbm,
             srcb, dstb, expb, eidc,
             locr0, locr1, locr2, gbuf0, gbuf1, gbuf2, acc_s,
             sg0, sg1, sg2, ss0, ss1, ss2):
    cid = lax.axis_index("c")
    sid = lax.axis_index("s")
    ebase = sid * EPB
    bufs = [gbuf0, gbuf1, gbuf2]
    locrs = [locr0, locr1, locr2]
    gsems = [sg0, sg1, sg2]
    ssems = [ss0, ss1, ss2]

    z16f = jnp.zeros((16,), jnp.float32)
    lane = jnp.arange(16, dtype=jnp.int32)

    def pass_body(p, _):
        lo = cid * (NPASS * PASS_ROWS) + p * PASS_ROWS
        hi = lo + PASS_ROWS

        for i in range(16):
            def zb_body(q, _):
                gbuf0[i, pl.ds(q * 16, 16)] = z16f
                return 0
            lax.fori_loop(0, HC // 16, zb_body, 0)
        for k in range(ROWS_PER_TILE // 16):
            pltpu.sync_copy(
                gbuf0, acc_s.at[pl.ds(sid * ROWS_PER_TILE + k * 16, 16), :])
        plsc.subcore_barrier()

        def seg_body(t, _):
            off = ebase + t * KB
            pltpu.sync_copy(src_hbm.at[pl.ds(off, KB)], srcb)
            pltpu.sync_copy(dst_hbm.at[pl.ds(off, KB)], dstb)
            for h in range(H):
                pltpu.sync_copy(expt_hbm.at[h, pl.ds(off, KB)],
                                expb.at[pl.ds(h * KB, KB)])

            def cmp_body(j, cnt):
                ds16 = pl.ds(j * 16, 16)
                sv = srcb[ds16]
                dv = dstb[ds16]
                m = (dv >= lo) & (dv < hi) & (sv >= 0)
                plsc.store_compressed(eidc.at[pl.ds(cnt, 16)],
                                      lane + j * 16, mask=m)
                return cnt + jnp.sum(m.astype(jnp.int32))
            cnt = lax.fori_loop(0, KB // 16, cmp_body, 0)

            z16i = jnp.zeros((16,), jnp.int32)
            eidc[pl.ds(cnt, 16)] = z16i
            eidc[pl.ds(cnt + 16, 16)] = z16i
            ntrip = (cnt + 15) // 16

            def prep(g, b):
                eidv = eidc[pl.ds(g * 16, 16)]
                srcv = jnp.maximum(plsc.load_gather(srcb, [eidv]), 0)
                pltpu.async_copy(xs_hbm.at[srcv], bufs[b], gsems[b])

            def swait(b):
                pltpu.make_async_copy(
                    bufs[b],
                    acc_s.at[plsc.Indices(locrs[b], ignored_value=-1)],
                    ssems[b]).wait()

            @pl.when(cnt > 0)
            def _():
                prep(0, 0)

                @pl.when(ntrip >= 2)
                def _():
                    prep(1, 1)

                def tri_body(i, _):
                    for b in range(3):
                        g = 3 * i + b

                        @pl.when(g < ntrip)
                        def _():
                            eidv = eidc[pl.ds(g * 16, 16)]
                            validv = (g * 16 + lane) < cnt
                            vmf = jnp.where(
                                validv, 1.0, 0.0).astype(jnp.float32)
                            srcv = jnp.maximum(
                                plsc.load_gather(srcb, [eidv]), 0)
                            pltpu.make_async_copy(
                                xs_hbm.at[srcv], bufs[b], gsems[b]).wait()
                            locv = jnp.where(
                                validv,
                                plsc.load_gather(dstb, [eidv]) - lo, -1)
                            locrs[b][pl.ds(0, 16)] = locv
                            buf = bufs[b]
                            for h in range(H):
                                expv = plsc.load_gather(
                                    expb, [eidv + h * KB]) * vmf
                                for e in range(16):
                                    av = z16f + expv[e]
                                    for q in range(C // 16):
                                        sl = pl.ds(h * C + q * 16, 16)
                                        buf[e, sl] = buf[e, sl] * av
                            pltpu.async_copy(
                                buf,
                                acc_s.at[plsc.Indices(locrs[b],
                                                      ignored_value=-1)],
                                ssems[b], add=True)
                            b2 = (b + 2) % 3

                            @pl.when(g >= 1)
                            def _():
                                swait(b2)

                            @pl.when(g + 2 < ntrip)
                            def _():
                                prep(g + 2, b2)
                    return 0
                lax.fori_loop(0, (ntrip + 2) // 3, tri_body, 0)
                for b in range(3):
                    @pl.when((ntrip - 1) % 3 == b)
                    def _():
                        swait(b)
            return 0
        lax.fori_loop(0, EPB // KB, seg_body, 0)

        plsc.subcore_barrier()
        row0 = lo + sid * ROWS_PER_TILE
        pltpu.sync_copy(
            acc_s.at[pl.ds(sid * ROWS_PER_TILE, ROWS_PER_TILE), :],
            msg_hbm.at[pl.ds(row0, ROWS_PER_TILE), :])
        return 0
    lax.fori_loop(0, NPASS, pass_body, 0)


def _k5(xs, srcp, dstp, expt):
    return pl.kernel(
        _k5_body,
        out_type=jax.ShapeDtypeStruct((MSGPAD, HC), jnp.float32),
        mesh=_mesh(),
        compiler_params=_sc_params,
        scratch_types=[
            pltpu.VMEM((KB,), jnp.int32),
            pltpu.VMEM((KB,), jnp.int32),
            pltpu.VMEM((H * KB,), jnp.float32),
            pltpu.VMEM((KB + 64,), jnp.int32),
            pltpu.VMEM((16,), jnp.int32),
            pltpu.VMEM((16,), jnp.int32),
            pltpu.VMEM((16,), jnp.int32),
            pltpu.VMEM((16, HC), jnp.float32),
            pltpu.VMEM((16, HC), jnp.float32),
            pltpu.VMEM((16, HC), jnp.float32),
            pltpu.VMEM_SHARED((PASS_ROWS, HC), jnp.float32),
            pltpu.SemaphoreType.DMA,
            pltpu.SemaphoreType.DMA,
            pltpu.SemaphoreType.DMA,
            pltpu.SemaphoreType.DMA,
            pltpu.SemaphoreType.DMA,
            pltpu.SemaphoreType.DMA,
        ],
    )(xs, srcp, dstp, expt)


def _k6_body(msg_ref, xs_ref, asrc_ref, adst_ref, ta_ref, tb_ref,
             bias_ref, w2_ref, b2_ref, out_ref):
    deg = ta_ref[:, 0:1] + tb_ref[:, 0:1]
    degc = jnp.maximum(deg, 1.0)
    gs = []
    for h in range(H):
        aes = ta_ref[:, 1 + h:2 + h] + tb_ref[:, 1 + h:2 + h]
        dnp = ta_ref[:, 5 + h:6 + h] + tb_ref[:, 5 + h:6 + h]
        ael = aes / degc
        ll = asrc_ref[:, h:h + 1] + adst_ref[:, h:h + 1] + ael
        ll = jnp.maximum(ll, 0.2 * ll)
        exl = jnp.exp(ll)
        den = dnp + exl + 1e-16
        gh = (msg_ref[:, h * C:(h + 1) * C]
              + exl * xs_ref[:, h * C:(h + 1) * C]) / den
        gs.append(gh)
    gat = jnp.concatenate(gs, axis=1) + bias_ref[...]
    out_ref[...] = (jnp.dot(gat, w2_ref[...], preferred_element_type=jnp.float32)
                    + b2_ref[...])


def _k6(msg, xs, asrc, adst, ta, tb, bias, w2, b2):
    n = msg.shape[0]
    f = w2.shape[1]
    bn = 256
    grid = (n + bn - 1) // bn
    return pl.pallas_call(
        _k6_body,
        grid=(grid,),
        in_specs=[
            pl.BlockSpec((bn, HC), lambda b: (b, 0)),
            pl.BlockSpec((bn, HC), lambda b: (b, 0)),
            pl.BlockSpec((bn, H), lambda b: (b, 0)),
            pl.BlockSpec((bn, H), lambda b: (b, 0)),
            pl.BlockSpec((bn, TROWS), lambda b: (b, 0)),
            pl.BlockSpec((bn, TROWS), lambda b: (b, 0)),
            pl.BlockSpec((1, HC), lambda b: (0, 0)),
            pl.BlockSpec((HC, f), lambda b: (0, 0)),
            pl.BlockSpec((1, f), lambda b: (0, 0)),
        ],
        out_specs=pl.BlockSpec((bn, f), lambda b: (b, 0)),
        out_shape=jax.ShapeDtypeStruct((n, f), jnp.float32),
    )(msg, xs, asrc, adst, ta, tb, bias, w2, b2)


def kernel(x, edge_index, edge_attr, W, att_src, att_dst, att_edge,
           We, bias_gat, W2, b2):
    src = edge_index[0].astype(jnp.int32)
    dst = edge_index[1].astype(jnp.int32)
    npad = EPAD - src.shape[0]
    srcp = jnp.concatenate([src, jnp.full((npad,), -1, jnp.int32)])
    dstp = jnp.concatenate([dst, jnp.zeros((npad,), jnp.int32)])

    xs, asrc, adst = _k1(x, W, att_src, att_dst)
    aet = _k1b(edge_attr, We, att_edge)
    aetp = jnp.concatenate([aet, jnp.zeros((H, npad), jnp.float32)], axis=1)

    expt, tbl = _k2(asrc.reshape(-1), adst.reshape(-1), srcp, dstp, aetp)
    msg = _k5(xs, srcp, dstp, expt)[:NNODE]

    ta = tbl[0, :TFLAT].reshape(TROWS, NNODE).T
    tb = tbl[1, :TFLAT].reshape(TROWS, NNODE).T
    return _k6(msg, xs, asrc, adst, ta, tb,
               bias_gat.reshape(1, HC), W2, b2.reshape(1, -1))


# speedup vs baseline: 16.6706x; 1.0181x over previous
import functools

import jax
import jax.numpy as jnp
from jax import lax
from jax.experimental import pallas as pl
from jax.experimental.pallas import tpu as pltpu
from jax.experimental.pallas import tpu_sc as plsc

H = 4
C = 256
HC = H * C
NNODE = 10000
NEDGE = 160000
EPAD = 163840
BATCH = 512

TROWS = 9
TFLAT = TROWS * NNODE
TPAD = 90112
TSTRIPE = TPAD // 16

PASS_ROWS = 1024
NPASS = 5
MSGPAD = 2 * NPASS * PASS_ROWS
ROWS_PER_TILE = PASS_ROWS // 16
EPB = EPAD // 16
KB = 2048

_mesh = functools.partial(
    plsc.VectorSubcoreMesh, core_axis_name="c", subcore_axis_name="s",
    num_cores=2, num_subcores=16)

_sc_params = pltpu.CompilerParams(needs_layout_passes=False,
                                  use_tc_tiling_on_sc=False)


def _k1_body(x_ref, w_ref, asw_ref, adw_ref, xs_ref, asrc_ref, adst_ref):
    xs = jnp.dot(x_ref[...], w_ref[...], preferred_element_type=jnp.float32)
    xs_ref[...] = xs
    acs = []
    acd = []
    for h in range(H):
        sl = xs[:, h * C:(h + 1) * C]
        acs.append(jnp.sum(sl * asw_ref[h:h + 1, :], axis=1, keepdims=True))
        acd.append(jnp.sum(sl * adw_ref[h:h + 1, :], axis=1, keepdims=True))
    asrc_ref[...] = jnp.concatenate(acs, axis=1)
    adst_ref[...] = jnp.concatenate(acd, axis=1)


def _k1(x, w, att_src, att_dst):
    n, f = x.shape
    bn = 256
    grid = (n + bn - 1) // bn
    return pl.pallas_call(
        _k1_body,
        grid=(grid,),
        in_specs=[
            pl.BlockSpec((bn, f), lambda b: (b, 0)),
            pl.BlockSpec((f, HC), lambda b: (0, 0)),
            pl.BlockSpec((H, C), lambda b: (0, 0)),
            pl.BlockSpec((H, C), lambda b: (0, 0)),
        ],
        out_specs=[
            pl.BlockSpec((bn, HC), lambda b: (b, 0)),
            pl.BlockSpec((bn, H), lambda b: (b, 0)),
            pl.BlockSpec((bn, H), lambda b: (b, 0)),
        ],
        out_shape=[
            jax.ShapeDtypeStruct((n, HC), jnp.float32),
            jax.ShapeDtypeStruct((n, H), jnp.float32),
            jax.ShapeDtypeStruct((n, H), jnp.float32),
        ],
    )(x, w, att_src, att_dst)


def _k1b_body(ea_ref, we_ref, aew_ref, aet_ref):
    cols = []
    for h in range(H):
        cols.append(jnp.sum(we_ref[:, h * C:(h + 1) * C] * aew_ref[h:h + 1, :],
                            axis=1, keepdims=True))
    mt = jnp.concatenate(cols, axis=1)
    aet_ref[...] = lax.dot_general(
        mt, ea_ref[...], (((0,), (1,)), ((), ())),
        preferred_element_type=jnp.float32)


def _k1b(edge_attr, we, att_edge):
    e, ed = edge_attr.shape
    be = 640
    grid = e // be
    return pl.pallas_call(
        _k1b_body,
        grid=(grid,),
        in_specs=[
            pl.BlockSpec((be, ed), lambda b: (b, 0)),
            pl.BlockSpec((ed, HC), lambda b: (0, 0)),
            pl.BlockSpec((H, C), lambda b: (0, 0)),
        ],
        out_specs=pl.BlockSpec((H, be), lambda b: (0, b)),
        out_shape=jax.ShapeDtypeStruct((H, e), jnp.float32),
    )(edge_attr, we, att_edge)


def _k2_body(asrc_hbm, adst_hbm, src_hbm, dst_hbm, aet_hbm,
             expt_hbm, tbl_hbm,
             asrc_v, adst_v, srcb, dstb, aeb, expb, valb,
             idx0, idx1, idx2, idx3, idx4, idx5, idx6, idx7, idx8,
             zerob, tbl_s):
    cid = lax.axis_index("c")
    sid = lax.axis_index("s")
    wid = cid * 16 + sid
    idxs = [idx0, idx1, idx2, idx3, idx4, idx5, idx6, idx7, idx8]

    pltpu.sync_copy(asrc_hbm, asrc_v)
    pltpu.sync_copy(adst_hbm, adst_v)

    z16f = jnp.zeros((16,), jnp.float32)

    def zero_body(i, _):
        zerob[pl.ds(i * 16, 16)] = z16f
        return 0
    lax.fori_loop(0, BATCH // 16, zero_body, 0)
    for i in range(TSTRIPE // BATCH):
        pltpu.sync_copy(zerob, tbl_s.at[pl.ds(sid * TSTRIPE + i * BATCH, BATCH)])
    plsc.subcore_barrier()

    ebase = wid * (EPAD // 32)

    def batch_body(b, _):
        off = ebase + b * BATCH
        pltpu.sync_copy(src_hbm.at[pl.ds(off, BATCH)], srcb)
        pltpu.sync_copy(dst_hbm.at[pl.ds(off, BATCH)], dstb)
        for h in range(H):
            pltpu.sync_copy(aet_hbm.at[h, pl.ds(off, BATCH)],
                            aeb.at[pl.ds(h * BATCH, BATCH)])

        def chunk_body(j, _):
            ds16 = pl.ds(j * 16, 16)
            sv = srcb[ds16]
            dv = dstb[ds16]
            m = sv >= 0
            svc = jnp.maximum(sv, 0)
            mf = jnp.where(m, 1.0, 0.0).astype(jnp.float32)
            valb[ds16] = mf
            for k in range(TROWS):
                idxs[k][ds16] = dv * TROWS + k
            for h in range(H):
                g1 = plsc.load_gather(asrc_v, [svc * H + h])
                g2 = plsc.load_gather(adst_v, [dv * H + h])
                av = aeb[pl.ds(h * BATCH + j * 16, 16)]
                l = g1 + g2 + av
                l = jnp.maximum(l, 0.2 * l)
                ex = jnp.exp(l) * mf
                expb[pl.ds(h * BATCH + j * 16, 16)] = ex
                valb[pl.ds((1 + h) * BATCH + j * 16, 16)] = av
                valb[pl.ds((5 + h) * BATCH + j * 16, 16)] = ex
            return 0
        lax.fori_loop(0, BATCH // 16, chunk_body, 0)

        for k in range(TROWS):
            pltpu.sync_copy(valb.at[pl.ds(k * BATCH, BATCH)],
                            tbl_s.at[idxs[k]], add=True)
        for h in range(H):
            pltpu.sync_copy(expb.at[pl.ds(h * BATCH, BATCH)],
                            expt_hbm.at[h, pl.ds(off, BATCH)])
        return 0
    lax.fori_loop(0, EPAD // 32 // BATCH, batch_body, 0)

    plsc.subcore_barrier()
    pltpu.sync_copy(tbl_s.at[pl.ds(sid * TSTRIPE, TSTRIPE)],
                    tbl_hbm.at[cid, pl.ds(sid * TSTRIPE, TSTRIPE)])


def _k2(asrc, adst, srcp, dstp, aetp):
    return pl.kernel(
        _k2_body,
        out_type=[
            jax.ShapeDtypeStruct((H, EPAD), jnp.float32),
            jax.ShapeDtypeStruct((2, TPAD), jnp.float32),
        ],
        mesh=_mesh(),
        compiler_params=_sc_params,
        scratch_types=[
            pltpu.VMEM((NNODE * H,), jnp.float32),
            pltpu.VMEM((NNODE * H,), jnp.float32),
            pltpu.VMEM((BATCH,), jnp.int32),
            pltpu.VMEM((BATCH,), jnp.int32),
            pltpu.VMEM((H * BATCH,), jnp.float32),
            pltpu.VMEM((H * BATCH,), jnp.float32),
            pltpu.VMEM((TROWS * BATCH,), jnp.float32),
        ] + [pltpu.VMEM((BATCH,), jnp.int32) for _ in range(TROWS)]
        + [
            pltpu.VMEM((BATCH,), jnp.float32),
            pltpu.VMEM_SHARED((TPAD,), jnp.float32),
        ],
    )(asrc, adst, srcp, dstp, aetp)


def _k5_body(xs_hbm, src_hbm, dst_hbm, expt_hbm, msg_hbm,
             srcb, dstb, expb, eidc,
             locr0, locr1, locr2, gbuf0, gbuf1, gbuf2, acc_s,
             sg0, sg1, sg2, ss0, ss1, ss2):
    cid = lax.axis_index("c")
    sid = lax.axis_index("s")
    ebase = sid * EPB
    bufs = [gbuf0, gbuf1, gbuf2]
    locrs = [locr0, locr1, locr2]
    gsems = [sg0, sg1, sg2]
    ssems = [ss0, ss1, ss2]

    z16f = jnp.zeros((16,), jnp.float32)
    lane = jnp.arange(16, dtype=jnp.int32)

    def pass_body(p, _):
        lo = cid * (NPASS * PASS_ROWS) + p * PASS_ROWS
        hi = lo + PASS_ROWS

        for i in range(16):
            def zb_body(q, _):
                gbuf0[i, pl.ds(q * 16, 16)] = z16f
                return 0
            lax.fori_loop(0, HC // 16, zb_body, 0)
        for k in range(ROWS_PER_TILE // 16):
            pltpu.sync_copy(
                gbuf0, acc_s.at[pl.ds(sid * ROWS_PER_TILE + k * 16, 16), :])
        plsc.subcore_barrier()

        def seg_body(t, _):
            off = ebase + t * KB
            pltpu.sync_copy(src_hbm.at[pl.ds(off, KB)], srcb)
            pltpu.sync_copy(dst_hbm.at[pl.ds(off, KB)], dstb)
            for h in range(H):
                pltpu.sync_copy(expt_hbm.at[h, pl.ds(off, KB)],
                                expb.at[pl.ds(h * KB, KB)])

            def cmp_body(j, cnt):
                ds16 = pl.ds(j * 16, 16)
                sv = srcb[ds16]
                dv = dstb[ds16]
                m = (dv >= lo) & (dv < hi) & (sv >= 0)
                plsc.store_compressed(eidc.at[pl.ds(cnt, 16)],
                                      lane + j * 16, mask=m)
                return cnt + jnp.sum(m.astype(jnp.int32))
            cnt = lax.fori_loop(0, KB // 16, cmp_body, 0)

            z16i = jnp.zeros((16,), jnp.int32)
            eidc[pl.ds(cnt, 16)] = z16i
            eidc[pl.ds(cnt + 16, 16)] = z16i
            ntrip = (cnt + 15) // 16

            def prep(g, b):
                eidv = eidc[pl.ds(g * 16, 16)]
                srcv = jnp.maximum(plsc.load_gather(srcb, [eidv]), 0)
                pltpu.async_copy(xs_hbm.at[srcv], bufs[b], gsems[b])

            def swait(b):
                pltpu.make_async_copy(
                    bufs[b],
                    acc_s.at[plsc.Indices(locrs[b], ignored_value=-1)],
                    ssems[b]).wait()

            @pl.when(cnt > 0)
            def _():
                prep(0, 0)

                @pl.when(ntrip >= 2)
                def _():
                    prep(1, 1)

                def tri_body(i, _):
                    for b in range(3):
                        g = 3 * i + b

                        @pl.when(g < ntrip)
                        def _():
                            eidv = eidc[pl.ds(g * 16, 16)]
                            validv = (g * 16 + lane) < cnt
                            vmf = jnp.where(
                                validv, 1.0, 0.0).astype(jnp.float32)
                            srcv = jnp.maximum(
                                plsc.load_gather(srcb, [eidv]), 0)
                            pltpu.make_async_copy(
                                xs_hbm.at[srcv], bufs[b], gsems[b]).wait()
                            locv = jnp.where(
                                validv,
                                plsc.load_gather(dstb, [eidv]) - lo, -1)
                            locrs[b][pl.ds(0, 16)] = locv
                            buf = bufs[b]
                            for h in range(H):
                                expv = plsc.load_gather(
                                    expb, [eidv + h * KB]) * vmf
                                for e in range(16):
                                    av = z16f + expv[e]
                                    for q in range(C // 16):
                                        sl = pl.ds(h * C + q * 16, 16)
                                        buf[e, sl] = buf[e, sl] * av
                            pltpu.async_copy(
                                buf,
                                acc_s.at[plsc.Indices(locrs[b],
                                                      ignored_value=-1)],
                                ssems[b], add=True)
                            b2 = (b + 2) % 3

                            @pl.when(g >= 1)
                            def _():
                                swait(b2)

                            @pl.when(g + 2 < ntrip)
                            def _():
                                prep(g + 2, b2)
                    return 0
                lax.fori_loop(0, (ntrip + 2) // 3, tri_body, 0)
                for b in range(3):
                    @pl.when((ntrip - 1) % 3 == b)
                    def _():
                        swait(b)
            return 0
        lax.fori_loop(0, EPB // KB, seg_body, 0)

        plsc.subcore_barrier()
        row0 = lo + sid * ROWS_PER_TILE
        pltpu.sync_copy(
            acc_s.at[pl.ds(sid * ROWS_PER_TILE, ROWS_PER_TILE), :],
            msg_hbm.at[pl.ds(row0, ROWS_PER_TILE), :])
        return 0
    lax.fori_loop(0, NPASS, pass_body, 0)


def _k5(xs, srcp, dstp, expt):
    return pl.kernel(
        _k5_body,
        out_type=jax.ShapeDtypeStruct((MSGPAD, HC), jnp.float32),
        mesh=_mesh(),
        compiler_params=_sc_params,
        scratch_types=[
            pltpu.VMEM((KB,), jnp.int32),
            pltpu.VMEM((KB,), jnp.int32),
            pltpu.VMEM((H * KB,), jnp.float32),
            pltpu.VMEM((KB + 64,), jnp.int32),
            pltpu.VMEM((16,), jnp.int32),
            pltpu.VMEM((16,), jnp.int32),
            pltpu.VMEM((16,), jnp.int32),
            pltpu.VMEM((16, HC), jnp.float32),
            pltpu.VMEM((16, HC), jnp.float32),
            pltpu.VMEM((16, HC), jnp.float32),
            pltpu.VMEM_SHARED((PASS_ROWS, HC), jnp.float32),
            pltpu.SemaphoreType.DMA,
            pltpu.SemaphoreType.DMA,
            pltpu.SemaphoreType.DMA,
            pltpu.SemaphoreType.DMA,
            pltpu.SemaphoreType.DMA,
            pltpu.SemaphoreType.DMA,
        ],
    )(xs, srcp, dstp, expt)


def _k6_body(msg_ref, xs_ref, asrc_ref, adst_ref, ta_ref, tb_ref,
             bias_ref, w2_ref, b2_ref, out_ref):
    deg = ta_ref[:, 0:1] + tb_ref[:, 0:1]
    degc = jnp.maximum(deg, 1.0)
    gs = []
    for h in range(H):
        aes = ta_ref[:, 1 + h:2 + h] + tb_ref[:, 1 + h:2 + h]
        dnp = ta_ref[:, 5 + h:6 + h] + tb_ref[:, 5 + h:6 + h]
        ael = aes / degc
        ll = asrc_ref[:, h:h + 1] + adst_ref[:, h:h + 1] + ael
        ll = jnp.maximum(ll, 0.2 * ll)
        exl = jnp.exp(ll)
        den = dnp + exl + 1e-16
        gh = (msg_ref[:, h * C:(h + 1) * C]
              + exl * xs_ref[:, h * C:(h + 1) * C]) / den
        gs.append(gh)
    gat = jnp.concatenate(gs, axis=1) + bias_ref[...]
    out_ref[...] = (jnp.dot(gat, w2_ref[...], preferred_element_type=jnp.float32)
                    + b2_ref[...])


def _k6(msg, xs, asrc, adst, ta, tb, bias, w2, b2):
    n = xs.shape[0]
    f = w2.shape[1]
    bn = 256
    grid = (n + bn - 1) // bn
    return pl.pallas_call(
        _k6_body,
        grid=(grid,),
        in_specs=[
            pl.BlockSpec((bn, HC), lambda b: (b, 0)),
            pl.BlockSpec((bn, HC), lambda b: (b, 0)),
            pl.BlockSpec((bn, H), lambda b: (b, 0)),
            pl.BlockSpec((bn, H), lambda b: (b, 0)),
            pl.BlockSpec((bn, TROWS), lambda b: (b, 0)),
            pl.BlockSpec((bn, TROWS), lambda b: (b, 0)),
            pl.BlockSpec((1, HC), lambda b: (0, 0)),
            pl.BlockSpec((HC, f), lambda b: (0, 0)),
            pl.BlockSpec((1, f), lambda b: (0, 0)),
        ],
        out_specs=pl.BlockSpec((bn, f), lambda b: (b, 0)),
        out_shape=jax.ShapeDtypeStruct((n, f), jnp.float32),
    )(msg, xs, asrc, adst, ta, tb, bias, w2, b2)


def kernel(x, edge_index, edge_attr, W, att_src, att_dst, att_edge,
           We, bias_gat, W2, b2):
    src = edge_index[0].astype(jnp.int32)
    dst = edge_index[1].astype(jnp.int32)
    npad = EPAD - src.shape[0]
    srcp = jnp.concatenate([src, jnp.full((npad,), -1, jnp.int32)])
    dstp = jnp.concatenate([dst, jnp.zeros((npad,), jnp.int32)])

    xs, asrc, adst = _k1(x, W, att_src, att_dst)
    aet = _k1b(edge_attr, We, att_edge)
    aetp = jnp.concatenate([aet, jnp.zeros((H, npad), jnp.float32)], axis=1)

    expt, tbl = _k2(asrc.reshape(-1), adst.reshape(-1), srcp, dstp, aetp)
    msg = _k5(xs, srcp, dstp, expt)

    ta = tbl[0, :TFLAT].reshape(NNODE, TROWS)
    tb = tbl[1, :TFLAT].reshape(NNODE, TROWS)
    return _k6(msg, xs, asrc, adst, ta, tb,
               bias_gat.reshape(1, HC), W2, b2.reshape(1, -1))


# speedup vs baseline: 16.7263x; 1.0033x over previous
import functools

import jax
import jax.numpy as jnp
from jax import lax
from jax.experimental import pallas as pl
from jax.experimental.pallas import tpu as pltpu
from jax.experimental.pallas import tpu_sc as plsc

H = 4
C = 256
HC = H * C
NNODE = 10000
NEDGE = 160000
EPAD = 163840
BATCH = 512

TROWS = 9
TFLAT = TROWS * NNODE
TPAD = 90112
TSTRIPE = TPAD // 16

PASS_ROWS = 1024
NPASS = 5
MSGPAD = 2 * NPASS * PASS_ROWS
ROWS_PER_TILE = PASS_ROWS // 16
EPB = EPAD // 16
KB = 2048

_mesh = functools.partial(
    plsc.VectorSubcoreMesh, core_axis_name="c", subcore_axis_name="s",
    num_cores=2, num_subcores=16)

_sc_params = pltpu.CompilerParams(needs_layout_passes=False,
                                  use_tc_tiling_on_sc=False)


def _k1_body(x_ref, w_ref, asw_ref, adw_ref, xs_ref, asrc_ref, adst_ref):
    xs = jnp.dot(x_ref[...], w_ref[...], preferred_element_type=jnp.float32)
    xs_ref[...] = xs
    acs = []
    acd = []
    for h in range(H):
        sl = xs[:, h * C:(h + 1) * C]
        acs.append(jnp.sum(sl * asw_ref[h:h + 1, :], axis=1, keepdims=True))
        acd.append(jnp.sum(sl * adw_ref[h:h + 1, :], axis=1, keepdims=True))
    asrc_ref[...] = jnp.concatenate(acs, axis=1)
    adst_ref[...] = jnp.concatenate(acd, axis=1)


def _k1(x, w, att_src, att_dst):
    n, f = x.shape
    bn = 256
    grid = (n + bn - 1) // bn
    return pl.pallas_call(
        _k1_body,
        grid=(grid,),
        in_specs=[
            pl.BlockSpec((bn, f), lambda b: (b, 0)),
            pl.BlockSpec((f, HC), lambda b: (0, 0)),
            pl.BlockSpec((H, C), lambda b: (0, 0)),
            pl.BlockSpec((H, C), lambda b: (0, 0)),
        ],
        out_specs=[
            pl.BlockSpec((bn, HC), lambda b: (b, 0)),
            pl.BlockSpec((bn, H), lambda b: (b, 0)),
            pl.BlockSpec((bn, H), lambda b: (b, 0)),
        ],
        out_shape=[
            jax.ShapeDtypeStruct((n, HC), jnp.float32),
            jax.ShapeDtypeStruct((n, H), jnp.float32),
            jax.ShapeDtypeStruct((n, H), jnp.float32),
        ],
    )(x, w, att_src, att_dst)


def _k1b_body(ea_ref, we_ref, aew_ref, aet_ref):
    cols = []
    for h in range(H):
        cols.append(jnp.sum(we_ref[:, h * C:(h + 1) * C] * aew_ref[h:h + 1, :],
                            axis=1, keepdims=True))
    mt = jnp.concatenate(cols, axis=1)
    aet_ref[...] = lax.dot_general(
        mt, ea_ref[...], (((0,), (1,)), ((), ())),
        preferred_element_type=jnp.float32)


def _k1b(edge_attr, we, att_edge):
    e, ed = edge_attr.shape
    be = 640
    grid = e // be
    return pl.pallas_call(
        _k1b_body,
        grid=(grid,),
        in_specs=[
            pl.BlockSpec((be, ed), lambda b: (b, 0)),
            pl.BlockSpec((ed, HC), lambda b: (0, 0)),
            pl.BlockSpec((H, C), lambda b: (0, 0)),
        ],
        out_specs=pl.BlockSpec((H, be), lambda b: (0, b)),
        out_shape=jax.ShapeDtypeStruct((H, EPAD), jnp.float32),
    )(edge_attr, we, att_edge)


def _k2_body(asrc_hbm, adst_hbm, ei_hbm, aet_hbm,
             expt_hbm, tbl_hbm,
             asrc_v, adst_v, srcb, dstb, aeb, expb, valb,
             idx0, idx1, idx2, idx3, idx4, idx5, idx6, idx7, idx8,
             zerob, tbl_s):
    cid = lax.axis_index("c")
    sid = lax.axis_index("s")
    wid = cid * 16 + sid
    idxs = [idx0, idx1, idx2, idx3, idx4, idx5, idx6, idx7, idx8]

    pltpu.sync_copy(asrc_hbm, asrc_v)
    pltpu.sync_copy(adst_hbm, adst_v)

    z16f = jnp.zeros((16,), jnp.float32)

    def zero_body(i, _):
        zerob[pl.ds(i * 16, 16)] = z16f
        return 0
    lax.fori_loop(0, BATCH // 16, zero_body, 0)
    for i in range(TSTRIPE // BATCH):
        pltpu.sync_copy(zerob, tbl_s.at[pl.ds(sid * TSTRIPE + i * BATCH, BATCH)])
    plsc.subcore_barrier()

    ebase = wid * (EPAD // 32)

    def batch_body(b, _):
        off = ebase + b * BATCH
        pltpu.sync_copy(ei_hbm.at[0, pl.ds(off, BATCH)], srcb)
        pltpu.sync_copy(ei_hbm.at[1, pl.ds(off, BATCH)], dstb)
        for h in range(H):
            pltpu.sync_copy(aet_hbm.at[h, pl.ds(off, BATCH)],
                            aeb.at[pl.ds(h * BATCH, BATCH)])

        def chunk_body(j, _):
            ds16 = pl.ds(j * 16, 16)
            sv = srcb[ds16]
            dv = dstb[ds16]
            m = sv >= 0
            svc = jnp.maximum(sv, 0)
            dvc = jnp.maximum(dv, 0)
            mf = jnp.where(m, 1.0, 0.0).astype(jnp.float32)
            valb[ds16] = mf
            for k in range(TROWS):
                idxs[k][ds16] = dvc * TROWS + k
            for h in range(H):
                g1 = plsc.load_gather(asrc_v, [svc * H + h])
                g2 = plsc.load_gather(adst_v, [dvc * H + h])
                av = aeb[pl.ds(h * BATCH + j * 16, 16)] * mf
                l = g1 + g2 + av
                l = jnp.maximum(l, 0.2 * l)
                ex = jnp.exp(l) * mf
                expb[pl.ds(h * BATCH + j * 16, 16)] = ex
                valb[pl.ds((1 + h) * BATCH + j * 16, 16)] = av
                valb[pl.ds((5 + h) * BATCH + j * 16, 16)] = ex
            return 0
        lax.fori_loop(0, BATCH // 16, chunk_body, 0)

        for k in range(TROWS):
            pltpu.sync_copy(valb.at[pl.ds(k * BATCH, BATCH)],
                            tbl_s.at[idxs[k]], add=True)
        for h in range(H):
            pltpu.sync_copy(expb.at[pl.ds(h * BATCH, BATCH)],
                            expt_hbm.at[h, pl.ds(off, BATCH)])
        return 0
    lax.fori_loop(0, EPAD // 32 // BATCH, batch_body, 0)

    plsc.subcore_barrier()
    pltpu.sync_copy(tbl_s.at[pl.ds(sid * TSTRIPE, TSTRIPE)],
                    tbl_hbm.at[cid, pl.ds(sid * TSTRIPE, TSTRIPE)])


def _k2(asrc, adst, eip, aetp):
    return pl.kernel(
        _k2_body,
        out_type=[
            jax.ShapeDtypeStruct((H, EPAD), jnp.float32),
            jax.ShapeDtypeStruct((2, TPAD), jnp.float32),
        ],
        mesh=_mesh(),
        compiler_params=_sc_params,
        scratch_types=[
            pltpu.VMEM((NNODE * H,), jnp.float32),
            pltpu.VMEM((NNODE * H,), jnp.float32),
            pltpu.VMEM((BATCH,), jnp.int32),
            pltpu.VMEM((BATCH,), jnp.int32),
            pltpu.VMEM((H * BATCH,), jnp.float32),
            pltpu.VMEM((H * BATCH,), jnp.float32),
            pltpu.VMEM((TROWS * BATCH,), jnp.float32),
        ] + [pltpu.VMEM((BATCH,), jnp.int32) for _ in range(TROWS)]
        + [
            pltpu.VMEM((BATCH,), jnp.float32),
            pltpu.VMEM_SHARED((TPAD,), jnp.float32),
        ],
    )(asrc, adst, eip, aetp)


def _k5_body(xs_hbm, ei_hbm, expt_hbm, msg_hbm,
             srcb, dstb, expb, eidc,
             locr0, locr1, locr2, gbuf0, gbuf1, gbuf2, acc_s,
             sg0, sg1, sg2, ss0, ss1, ss2):
    cid = lax.axis_index("c")
    sid = lax.axis_index("s")
    ebase = sid * EPB
    bufs = [gbuf0, gbuf1, gbuf2]
    locrs = [locr0, locr1, locr2]
    gsems = [sg0, sg1, sg2]
    ssems = [ss0, ss1, ss2]

    z16f = jnp.zeros((16,), jnp.float32)
    lane = jnp.arange(16, dtype=jnp.int32)

    def pass_body(p, _):
        lo = cid * (NPASS * PASS_ROWS) + p * PASS_ROWS
        hi = lo + PASS_ROWS

        for i in range(16):
            def zb_body(q, _):
                gbuf0[i, pl.ds(q * 16, 16)] = z16f
                return 0
            lax.fori_loop(0, HC // 16, zb_body, 0)
        for k in range(ROWS_PER_TILE // 16):
            pltpu.sync_copy(
                gbuf0, acc_s.at[pl.ds(sid * ROWS_PER_TILE + k * 16, 16), :])
        plsc.subcore_barrier()

        def seg_body(t, _):
            off = ebase + t * KB
            pltpu.sync_copy(ei_hbm.at[0, pl.ds(off, KB)], srcb)
            pltpu.sync_copy(ei_hbm.at[1, pl.ds(off, KB)], dstb)
            for h in range(H):
                pltpu.sync_copy(expt_hbm.at[h, pl.ds(off, KB)],
                                expb.at[pl.ds(h * KB, KB)])

            def cmp_body(j, cnt):
                ds16 = pl.ds(j * 16, 16)
                sv = srcb[ds16]
                dv = dstb[ds16]
                m = (dv >= lo) & (dv < hi) & (sv >= 0)
                plsc.store_compressed(eidc.at[pl.ds(cnt, 16)],
                                      lane + j * 16, mask=m)
                return cnt + jnp.sum(m.astype(jnp.int32))
            cnt = lax.fori_loop(0, KB // 16, cmp_body, 0)

            z16i = jnp.zeros((16,), jnp.int32)
            eidc[pl.ds(cnt, 16)] = z16i
            eidc[pl.ds(cnt + 16, 16)] = z16i
            ntrip = (cnt + 15) // 16

            def prep(g, b):
                eidv = eidc[pl.ds(g * 16, 16)]
                srcv = jnp.maximum(plsc.load_gather(srcb, [eidv]), 0)
                pltpu.async_copy(xs_hbm.at[srcv], bufs[b], gsems[b])

            def swait(b):
                pltpu.make_async_copy(
                    bufs[b],
                    acc_s.at[plsc.Indices(locrs[b], ignored_value=-1)],
                    ssems[b]).wait()

            @pl.when(cnt > 0)
            def _():
                prep(0, 0)

                @pl.when(ntrip >= 2)
                def _():
                    prep(1, 1)

                def tri_body(i, _):
                    for b in range(3):
                        g = 3 * i + b

                        @pl.when(g < ntrip)
                        def _():
                            eidv = eidc[pl.ds(g * 16, 16)]
                            validv = (g * 16 + lane) < cnt
                            vmf = jnp.where(
                                validv, 1.0, 0.0).astype(jnp.float32)
                            srcv = jnp.maximum(
                                plsc.load_gather(srcb, [eidv]), 0)
                            pltpu.make_async_copy(
                                xs_hbm.at[srcv], bufs[b], gsems[b]).wait()
                            locv = jnp.where(
                                validv,
                                plsc.load_gather(dstb, [eidv]) - lo, -1)
                            locrs[b][pl.ds(0, 16)] = locv
                            buf = bufs[b]
                            for h in range(H):
                                expv = plsc.load_gather(
                                    expb, [eidv + h * KB]) * vmf
                                for e in range(16):
                                    av = z16f + expv[e]
                                    for q in range(C // 16):
                                        sl = pl.ds(h * C + q * 16, 16)
                                        buf[e, sl] = buf[e, sl] * av
                            pltpu.async_copy(
                                buf,
                                acc_s.at[plsc.Indices(locrs[b],
                                                      ignored_value=-1)],
                                ssems[b], add=True)
                            b2 = (b + 2) % 3

                            @pl.when(g >= 1)
                            def _():
                                swait(b2)

                            @pl.when(g + 2 < ntrip)
                            def _():
                                prep(g + 2, b2)
                    return 0
                lax.fori_loop(0, (ntrip + 2) // 3, tri_body, 0)
                for b in range(3):
                    @pl.when((ntrip - 1) % 3 == b)
                    def _():
                        swait(b)
            return 0
        lax.fori_loop(0, EPB // KB, seg_body, 0)

        plsc.subcore_barrier()
        row0 = lo + sid * ROWS_PER_TILE
        pltpu.sync_copy(
            acc_s.at[pl.ds(sid * ROWS_PER_TILE, ROWS_PER_TILE), :],
            msg_hbm.at[pl.ds(row0, ROWS_PER_TILE), :])
        return 0
    lax.fori_loop(0, NPASS, pass_body, 0)


def _k5(xs, eip, expt):
    return pl.kernel(
        _k5_body,
        out_type=jax.ShapeDtypeStruct((MSGPAD, HC), jnp.float32),
        mesh=_mesh(),
        compiler_params=_sc_params,
        scratch_types=[
            pltpu.VMEM((KB,), jnp.int32),
            pltpu.VMEM((KB,), jnp.int32),
            pltpu.VMEM((H * KB,), jnp.float32),
            pltpu.VMEM((KB + 64,), jnp.int32),
            pltpu.VMEM((16,), jnp.int32),
            pltpu.VMEM((16,), jnp.int32),
            pltpu.VMEM((16,), jnp.int32),
            pltpu.VMEM((16, HC), jnp.float32),
            pltpu.VMEM((16, HC), jnp.float32),
            pltpu.VMEM((16, HC), jnp.float32),
            pltpu.VMEM_SHARED((PASS_ROWS, HC), jnp.float32),
            pltpu.SemaphoreType.DMA,
            pltpu.SemaphoreType.DMA,
            pltpu.SemaphoreType.DMA,
            pltpu.SemaphoreType.DMA,
            pltpu.SemaphoreType.DMA,
            pltpu.SemaphoreType.DMA,
        ],
    )(xs, eip, expt)


def _k6_body(msg_ref, xs_ref, asrc_ref, adst_ref, ta_ref, tb_ref,
             bias_ref, w2_ref, b2_ref, out_ref):
    deg = ta_ref[:, 0:1] + tb_ref[:, 0:1]
    degc = jnp.maximum(deg, 1.0)
    gs = []
    for h in range(H):
        aes = ta_ref[:, 1 + h:2 + h] + tb_ref[:, 1 + h:2 + h]
        dnp = ta_ref[:, 5 + h:6 + h] + tb_ref[:, 5 + h:6 + h]
        ael = aes / degc
        ll = asrc_ref[:, h:h + 1] + adst_ref[:, h:h + 1] + ael
        ll = jnp.maximum(ll, 0.2 * ll)
        exl = jnp.exp(ll)
        den = dnp + exl + 1e-16
        gh = (msg_ref[:, h * C:(h + 1) * C]
              + exl * xs_ref[:, h * C:(h + 1) * C]) / den
        gs.append(gh)
    gat = jnp.concatenate(gs, axis=1) + bias_ref[...]
    out_ref[...] = (jnp.dot(gat, w2_ref[...], preferred_element_type=jnp.float32)
                    + b2_ref[...])


def _k6(msg, xs, asrc, adst, ta, tb, bias, w2, b2):
    n = xs.shape[0]
    f = w2.shape[1]
    bn = 256
    grid = (n + bn - 1) // bn
    return pl.pallas_call(
        _k6_body,
        grid=(grid,),
        in_specs=[
            pl.BlockSpec((bn, HC), lambda b: (b, 0)),
            pl.BlockSpec((bn, HC), lambda b: (b, 0)),
            pl.BlockSpec((bn, H), lambda b: (b, 0)),
            pl.BlockSpec((bn, H), lambda b: (b, 0)),
            pl.BlockSpec((bn, TROWS), lambda b: (b, 0)),
            pl.BlockSpec((bn, TROWS), lambda b: (b, 0)),
            pl.BlockSpec((1, HC), lambda b: (0, 0)),
            pl.BlockSpec((HC, f), lambda b: (0, 0)),
            pl.BlockSpec((1, f), lambda b: (0, 0)),
        ],
        out_specs=pl.BlockSpec((bn, f), lambda b: (b, 0)),
        out_shape=jax.ShapeDtypeStruct((n, f), jnp.float32),
    )(msg, xs, asrc, adst, ta, tb, bias, w2, b2)


def kernel(x, edge_index, edge_attr, W, att_src, att_dst, att_edge,
           We, bias_gat, W2, b2):
    npad = EPAD - edge_index.shape[1]
    eip = jnp.concatenate(
        [edge_index.astype(jnp.int32),
         jnp.full((2, npad), -1, jnp.int32)], axis=1)

    xs, asrc, adst = _k1(x, W, att_src, att_dst)
    aetp = _k1b(edge_attr, We, att_edge)

    expt, tbl = _k2(asrc.reshape(-1), adst.reshape(-1), eip, aetp)
    msg = _k5(xs, eip, expt)

    ta = tbl[0, :TFLAT].reshape(NNODE, TROWS)
    tb = tbl[1, :TFLAT].reshape(NNODE, TROWS)
    return _k6(msg, xs, asrc, adst, ta, tb,
               bias_gat.reshape(1, HC), W2, b2.reshape(1, -1))


# speedup vs baseline: 17.0633x; 1.0202x over previous
import functools

import jax
import jax.numpy as jnp
from jax import lax
from jax.experimental import pallas as pl
from jax.experimental.pallas import tpu as pltpu
from jax.experimental.pallas import tpu_sc as plsc

H = 4
C = 256
HC = H * C
NNODE = 10000
NEDGE = 160000
EPAD = 163840
BATCH = 512

TROWS = 9
TFLAT = TROWS * NNODE
TPAD = 90112
TSTRIPE = TPAD // 16

PASS_ROWS = 1024
NPASS = 5
MSGPAD = 2 * NPASS * PASS_ROWS
ROWS_PER_TILE = PASS_ROWS // 16
EPB = EPAD // 16
KB = 2048

_mesh = functools.partial(
    plsc.VectorSubcoreMesh, core_axis_name="c", subcore_axis_name="s",
    num_cores=2, num_subcores=16)

_sc_params = pltpu.CompilerParams(needs_layout_passes=False,
                                  use_tc_tiling_on_sc=False)


def _k1_body(x_ref, w_ref, asw_ref, adw_ref, xs_ref, asrc_ref, adst_ref):
    xs = jnp.dot(x_ref[...], w_ref[...], preferred_element_type=jnp.float32)
    xs_ref[...] = xs
    acs = []
    acd = []
    for h in range(H):
        sl = xs[:, h * C:(h + 1) * C]
        acs.append(jnp.sum(sl * asw_ref[h:h + 1, :], axis=1, keepdims=True))
        acd.append(jnp.sum(sl * adw_ref[h:h + 1, :], axis=1, keepdims=True))
    asrc_ref[...] = jnp.concatenate(acs, axis=1)
    adst_ref[...] = jnp.concatenate(acd, axis=1)


def _k1(x, w, att_src, att_dst):
    n, f = x.shape
    bn = 512
    grid = (n + bn - 1) // bn
    return pl.pallas_call(
        _k1_body,
        grid=(grid,),
        in_specs=[
            pl.BlockSpec((bn, f), lambda b: (b, 0)),
            pl.BlockSpec((f, HC), lambda b: (0, 0)),
            pl.BlockSpec((H, C), lambda b: (0, 0)),
            pl.BlockSpec((H, C), lambda b: (0, 0)),
        ],
        out_specs=[
            pl.BlockSpec((bn, HC), lambda b: (b, 0)),
            pl.BlockSpec((bn, H), lambda b: (b, 0)),
            pl.BlockSpec((bn, H), lambda b: (b, 0)),
        ],
        out_shape=[
            jax.ShapeDtypeStruct((n, HC), jnp.float32),
            jax.ShapeDtypeStruct((n, H), jnp.float32),
            jax.ShapeDtypeStruct((n, H), jnp.float32),
        ],
    )(x, w, att_src, att_dst)


def _k1b_body(ea_ref, we_ref, aew_ref, aet_ref):
    cols = []
    for h in range(H):
        cols.append(jnp.sum(we_ref[:, h * C:(h + 1) * C] * aew_ref[h:h + 1, :],
                            axis=1, keepdims=True))
    mt = jnp.concatenate(cols, axis=1)
    aet_ref[...] = lax.dot_general(
        mt, ea_ref[...], (((0,), (1,)), ((), ())),
        preferred_element_type=jnp.float32)


def _k1b(edge_attr, we, att_edge):
    e, ed = edge_attr.shape
    be = 640
    grid = e // be
    return pl.pallas_call(
        _k1b_body,
        grid=(grid,),
        in_specs=[
            pl.BlockSpec((be, ed), lambda b: (b, 0)),
            pl.BlockSpec((ed, HC), lambda b: (0, 0)),
            pl.BlockSpec((H, C), lambda b: (0, 0)),
        ],
        out_specs=pl.BlockSpec((H, be), lambda b: (0, b)),
        out_shape=jax.ShapeDtypeStruct((H, EPAD), jnp.float32),
    )(edge_attr, we, att_edge)


def _k2_body(asrc_hbm, adst_hbm, ei_hbm, aet_hbm,
             expt_hbm, tbl_hbm,
             asrc_v, adst_v, srcb, dstb, aeb, expb, valb,
             idx0, idx1, idx2, idx3, idx4, idx5, idx6, idx7, idx8,
             zerob, tbl_s):
    cid = lax.axis_index("c")
    sid = lax.axis_index("s")
    wid = cid * 16 + sid
    idxs = [idx0, idx1, idx2, idx3, idx4, idx5, idx6, idx7, idx8]

    pltpu.sync_copy(asrc_hbm, asrc_v)
    pltpu.sync_copy(adst_hbm, adst_v)

    z16f = jnp.zeros((16,), jnp.float32)

    def zero_body(i, _):
        zerob[pl.ds(i * 16, 16)] = z16f
        return 0
    lax.fori_loop(0, BATCH // 16, zero_body, 0)
    for i in range(TSTRIPE // BATCH):
        pltpu.sync_copy(zerob, tbl_s.at[pl.ds(sid * TSTRIPE + i * BATCH, BATCH)])
    plsc.subcore_barrier()

    ebase = wid * (EPAD // 32)

    def batch_body(b, _):
        off = ebase + b * BATCH
        pltpu.sync_copy(ei_hbm.at[0, pl.ds(off, BATCH)], srcb)
        pltpu.sync_copy(ei_hbm.at[1, pl.ds(off, BATCH)], dstb)
        for h in range(H):
            pltpu.sync_copy(aet_hbm.at[h, pl.ds(off, BATCH)],
                            aeb.at[pl.ds(h * BATCH, BATCH)])

        def chunk_body(j, _):
            ds16 = pl.ds(j * 16, 16)
            sv = srcb[ds16]
            dv = dstb[ds16]
            m = sv >= 0
            svc = jnp.maximum(sv, 0)
            dvc = jnp.maximum(dv, 0)
            mf = jnp.where(m, 1.0, 0.0).astype(jnp.float32)
            valb[ds16] = mf
            for k in range(TROWS):
                idxs[k][ds16] = dvc * TROWS + k
            for h in range(H):
                g1 = plsc.load_gather(asrc_v, [svc * H + h])
                g2 = plsc.load_gather(adst_v, [dvc * H + h])
                av = aeb[pl.ds(h * BATCH + j * 16, 16)] * mf
                l = g1 + g2 + av
                l = jnp.maximum(l, 0.2 * l)
                ex = jnp.exp(l) * mf
                expb[pl.ds(h * BATCH + j * 16, 16)] = ex
                valb[pl.ds((1 + h) * BATCH + j * 16, 16)] = av
                valb[pl.ds((5 + h) * BATCH + j * 16, 16)] = ex
            return 0
        lax.fori_loop(0, BATCH // 16, chunk_body, 0)

        for k in range(TROWS):
            pltpu.sync_copy(valb.at[pl.ds(k * BATCH, BATCH)],
                            tbl_s.at[idxs[k]], add=True)
        for h in range(H):
            pltpu.sync_copy(expb.at[pl.ds(h * BATCH, BATCH)],
                            expt_hbm.at[h, pl.ds(off, BATCH)])
        return 0
    lax.fori_loop(0, EPAD // 32 // BATCH, batch_body, 0)

    plsc.subcore_barrier()
    pltpu.sync_copy(tbl_s.at[pl.ds(sid * TSTRIPE, TSTRIPE)],
                    tbl_hbm.at[cid, pl.ds(sid * TSTRIPE, TSTRIPE)])


def _k2(asrc, adst, eip, aetp):
    return pl.kernel(
        _k2_body,
        out_type=[
            jax.ShapeDtypeStruct((H, EPAD), jnp.float32),
            jax.ShapeDtypeStruct((2, TPAD), jnp.float32),
        ],
        mesh=_mesh(),
        compiler_params=_sc_params,
        scratch_types=[
            pltpu.VMEM((NNODE * H,), jnp.float32),
            pltpu.VMEM((NNODE * H,), jnp.float32),
            pltpu.VMEM((BATCH,), jnp.int32),
            pltpu.VMEM((BATCH,), jnp.int32),
            pltpu.VMEM((H * BATCH,), jnp.float32),
            pltpu.VMEM((H * BATCH,), jnp.float32),
            pltpu.VMEM((TROWS * BATCH,), jnp.float32),
        ] + [pltpu.VMEM((BATCH,), jnp.int32) for _ in range(TROWS)]
        + [
            pltpu.VMEM((BATCH,), jnp.float32),
            pltpu.VMEM_SHARED((TPAD,), jnp.float32),
        ],
    )(asrc, adst, eip, aetp)


def _k5_body(xs_hbm, ei_hbm, expt_hbm, msg_hbm,
             srcb, dstb, expb, eidc,
             locr0, locr1, locr2, gbuf0, gbuf1, gbuf2, acc_s,
             sg0, sg1, sg2, ss0, ss1, ss2):
    cid = lax.axis_index("c")
    sid = lax.axis_index("s")
    ebase = sid * EPB
    bufs = [gbuf0, gbuf1, gbuf2]
    locrs = [locr0, locr1, locr2]
    gsems = [sg0, sg1, sg2]
    ssems = [ss0, ss1, ss2]

    z16f = jnp.zeros((16,), jnp.float32)
    lane = jnp.arange(16, dtype=jnp.int32)

    def pass_body(p, _):
        lo = cid * (NPASS * PASS_ROWS) + p * PASS_ROWS
        hi = lo + PASS_ROWS

        for i in range(16):
            def zb_body(q, _):
                gbuf0[i, pl.ds(q * 16, 16)] = z16f
                return 0
            lax.fori_loop(0, HC // 16, zb_body, 0)
        for k in range(ROWS_PER_TILE // 16):
            pltpu.sync_copy(
                gbuf0, acc_s.at[pl.ds(sid * ROWS_PER_TILE + k * 16, 16), :])
        plsc.subcore_barrier()

        def seg_body(t, _):
            off = ebase + t * KB
            pltpu.sync_copy(ei_hbm.at[0, pl.ds(off, KB)], srcb)
            pltpu.sync_copy(ei_hbm.at[1, pl.ds(off, KB)], dstb)
            for h in range(H):
                pltpu.sync_copy(expt_hbm.at[h, pl.ds(off, KB)],
                                expb.at[pl.ds(h * KB, KB)])

            def cmp_body(j, cnt):
                ds16 = pl.ds(j * 16, 16)
                sv = srcb[ds16]
                dv = dstb[ds16]
                m = (dv >= lo) & (dv < hi) & (sv >= 0)
                plsc.store_compressed(eidc.at[pl.ds(cnt, 16)],
                                      lane + j * 16, mask=m)
                return cnt + jnp.sum(m.astype(jnp.int32))
            cnt = lax.fori_loop(0, KB // 16, cmp_body, 0)

            z16i = jnp.zeros((16,), jnp.int32)
            eidc[pl.ds(cnt, 16)] = z16i
            eidc[pl.ds(cnt + 16, 16)] = z16i
            ntrip = (cnt + 15) // 16

            def prep(g, b):
                eidv = eidc[pl.ds(g * 16, 16)]
                srcv = jnp.maximum(plsc.load_gather(srcb, [eidv]), 0)
                pltpu.async_copy(xs_hbm.at[srcv], bufs[b], gsems[b])

            def swait(b):
                pltpu.make_async_copy(
                    bufs[b],
                    acc_s.at[plsc.Indices(locrs[b], ignored_value=-1)],
                    ssems[b]).wait()

            @pl.when(cnt > 0)
            def _():
                prep(0, 0)

                @pl.when(ntrip >= 2)
                def _():
                    prep(1, 1)

                def tri_body(i, _):
                    for b in range(3):
                        g = 3 * i + b

                        @pl.when(g < ntrip)
                        def _():
                            eidv = eidc[pl.ds(g * 16, 16)]
                            validv = (g * 16 + lane) < cnt
                            vmf = jnp.where(
                                validv, 1.0, 0.0).astype(jnp.float32)
                            srcv = jnp.maximum(
                                plsc.load_gather(srcb, [eidv]), 0)
                            pltpu.make_async_copy(
                                xs_hbm.at[srcv], bufs[b], gsems[b]).wait()
                            locv = jnp.where(
                                validv,
                                plsc.load_gather(dstb, [eidv]) - lo, -1)
                            locrs[b][pl.ds(0, 16)] = locv
                            buf = bufs[b]
                            for h in range(H):
                                expv = plsc.load_gather(
                                    expb, [eidv + h * KB]) * vmf
                                for e in range(16):
                                    av = z16f + expv[e]
                                    for q in range(C // 16):
                                        sl = pl.ds(h * C + q * 16, 16)
                                        buf[e, sl] = buf[e, sl] * av
                            pltpu.async_copy(
                                buf,
                                acc_s.at[plsc.Indices(locrs[b],
                                                      ignored_value=-1)],
                                ssems[b], add=True)
                            b2 = (b + 2) % 3

                            @pl.when(g >= 1)
                            def _():
                                swait(b2)

                            @pl.when(g + 2 < ntrip)
                            def _():
                                prep(g + 2, b2)
                    return 0
                lax.fori_loop(0, (ntrip + 2) // 3, tri_body, 0)
                for b in range(3):
                    @pl.when((ntrip - 1) % 3 == b)
                    def _():
                        swait(b)
            return 0
        lax.fori_loop(0, EPB // KB, seg_body, 0)

        plsc.subcore_barrier()
        row0 = lo + sid * ROWS_PER_TILE
        pltpu.sync_copy(
            acc_s.at[pl.ds(sid * ROWS_PER_TILE, ROWS_PER_TILE), :],
            msg_hbm.at[pl.ds(row0, ROWS_PER_TILE), :])
        return 0
    lax.fori_loop(0, NPASS, pass_body, 0)


def _k5(xs, eip, expt):
    return pl.kernel(
        _k5_body,
        out_type=jax.ShapeDtypeStruct((MSGPAD, HC), jnp.float32),
        mesh=_mesh(),
        compiler_params=_sc_params,
        scratch_types=[
            pltpu.VMEM((KB,), jnp.int32),
            pltpu.VMEM((KB,), jnp.int32),
            pltpu.VMEM((H * KB,), jnp.float32),
            pltpu.VMEM((KB + 64,), jnp.int32),
            pltpu.VMEM((16,), jnp.int32),
            pltpu.VMEM((16,), jnp.int32),
            pltpu.VMEM((16,), jnp.int32),
            pltpu.VMEM((16, HC), jnp.float32),
            pltpu.VMEM((16, HC), jnp.float32),
            pltpu.VMEM((16, HC), jnp.float32),
            pltpu.VMEM_SHARED((PASS_ROWS, HC), jnp.float32),
            pltpu.SemaphoreType.DMA,
            pltpu.SemaphoreType.DMA,
            pltpu.SemaphoreType.DMA,
            pltpu.SemaphoreType.DMA,
            pltpu.SemaphoreType.DMA,
            pltpu.SemaphoreType.DMA,
        ],
    )(xs, eip, expt)


def _k6_body(msg_ref, xs_ref, asrc_ref, adst_ref, ta_ref, tb_ref,
             bias_ref, w2_ref, b2_ref, out_ref):
    deg = ta_ref[:, 0:1] + tb_ref[:, 0:1]
    degc = jnp.maximum(deg, 1.0)
    gs = []
    for h in range(H):
        aes = ta_ref[:, 1 + h:2 + h] + tb_ref[:, 1 + h:2 + h]
        dnp = ta_ref[:, 5 + h:6 + h] + tb_ref[:, 5 + h:6 + h]
        ael = aes / degc
        ll = asrc_ref[:, h:h + 1] + adst_ref[:, h:h + 1] + ael
        ll = jnp.maximum(ll, 0.2 * ll)
        exl = jnp.exp(ll)
        den = dnp + exl + 1e-16
        gh = (msg_ref[:, h * C:(h + 1) * C]
              + exl * xs_ref[:, h * C:(h + 1) * C]) / den
        gs.append(gh)
    gat = jnp.concatenate(gs, axis=1) + bias_ref[...]
    out_ref[...] = (jnp.dot(gat, w2_ref[...], preferred_element_type=jnp.float32)
                    + b2_ref[...])


def _k6(msg, xs, asrc, adst, ta, tb, bias, w2, b2):
    n = xs.shape[0]
    f = w2.shape[1]
    bn = 512
    grid = (n + bn - 1) // bn
    return pl.pallas_call(
        _k6_body,
        grid=(grid,),
        in_specs=[
            pl.BlockSpec((bn, HC), lambda b: (b, 0)),
            pl.BlockSpec((bn, HC), lambda b: (b, 0)),
            pl.BlockSpec((bn, H), lambda b: (b, 0)),
            pl.BlockSpec((bn, H), lambda b: (b, 0)),
            pl.BlockSpec((bn, TROWS), lambda b: (b, 0)),
            pl.BlockSpec((bn, TROWS), lambda b: (b, 0)),
            pl.BlockSpec((1, HC), lambda b: (0, 0)),
            pl.BlockSpec((HC, f), lambda b: (0, 0)),
            pl.BlockSpec((1, f), lambda b: (0, 0)),
        ],
        out_specs=pl.BlockSpec((bn, f), lambda b: (b, 0)),
        out_shape=jax.ShapeDtypeStruct((n, f), jnp.float32),
    )(msg, xs, asrc, adst, ta, tb, bias, w2, b2)


def kernel(x, edge_index, edge_attr, W, att_src, att_dst, att_edge,
           We, bias_gat, W2, b2):
    npad = EPAD - edge_index.shape[1]
    eip = jnp.concatenate(
        [edge_index.astype(jnp.int32),
         jnp.full((2, npad), -1, jnp.int32)], axis=1)

    xs, asrc, adst = _k1(x, W, att_src, att_dst)
    aetp = _k1b(edge_attr, We, att_edge)

    expt, tbl = _k2(asrc.reshape(-1), adst.reshape(-1), eip, aetp)
    msg = _k5(xs, eip, expt)

    ta = tbl[0, :TFLAT].reshape(NNODE, TROWS)
    tb = tbl[1, :TFLAT].reshape(NNODE, TROWS)
    return _k6(msg, xs, asrc, adst, ta, tb,
               bias_gat.reshape(1, HC), W2, b2.reshape(1, -1))


# speedup vs baseline: 17.1457x; 1.0048x over previous
import functools

import jax
import jax.numpy as jnp
from jax import lax
from jax.experimental import pallas as pl
from jax.experimental.pallas import tpu as pltpu
from jax.experimental.pallas import tpu_sc as plsc

H = 4
C = 256
HC = H * C
NNODE = 10000
NEDGE = 160000
EPAD = 163840
BATCH = 512

TROWS = 9
TFLAT = TROWS * NNODE
TPAD = 90112
TSTRIPE = TPAD // 16

PASS_ROWS = 1024
NPASS = 5
MSGPAD = 2 * NPASS * PASS_ROWS
ROWS_PER_TILE = PASS_ROWS // 16
EPB = EPAD // 16
KB = 2048

_mesh = functools.partial(
    plsc.VectorSubcoreMesh, core_axis_name="c", subcore_axis_name="s",
    num_cores=2, num_subcores=16)

_sc_params = pltpu.CompilerParams(needs_layout_passes=False,
                                  use_tc_tiling_on_sc=False)


def _k1_body(x_ref, w_ref, asw_ref, adw_ref, xs_ref, asrc_ref, adst_ref):
    xs = jnp.dot(x_ref[...], w_ref[...], preferred_element_type=jnp.float32)
    xs_ref[...] = xs
    acs = []
    acd = []
    for h in range(H):
        sl = xs[:, h * C:(h + 1) * C]
        acs.append(jnp.sum(sl * asw_ref[h:h + 1, :], axis=1, keepdims=True))
        acd.append(jnp.sum(sl * adw_ref[h:h + 1, :], axis=1, keepdims=True))
    asrc_ref[...] = jnp.concatenate(acs, axis=1)
    adst_ref[...] = jnp.concatenate(acd, axis=1)


def _k1(x, w, att_src, att_dst):
    n, f = x.shape
    bn = 1024
    grid = (n + bn - 1) // bn
    return pl.pallas_call(
        _k1_body,
        grid=(grid,),
        in_specs=[
            pl.BlockSpec((bn, f), lambda b: (b, 0)),
            pl.BlockSpec((f, HC), lambda b: (0, 0)),
            pl.BlockSpec((H, C), lambda b: (0, 0)),
            pl.BlockSpec((H, C), lambda b: (0, 0)),
        ],
        out_specs=[
            pl.BlockSpec((bn, HC), lambda b: (b, 0)),
            pl.BlockSpec((bn, H), lambda b: (b, 0)),
            pl.BlockSpec((bn, H), lambda b: (b, 0)),
        ],
        out_shape=[
            jax.ShapeDtypeStruct((n, HC), jnp.float32),
            jax.ShapeDtypeStruct((n, H), jnp.float32),
            jax.ShapeDtypeStruct((n, H), jnp.float32),
        ],
    )(x, w, att_src, att_dst)


def _k1b_body(ea_ref, we_ref, aew_ref, aet_ref):
    cols = []
    for h in range(H):
        cols.append(jnp.sum(we_ref[:, h * C:(h + 1) * C] * aew_ref[h:h + 1, :],
                            axis=1, keepdims=True))
    mt = jnp.concatenate(cols, axis=1)
    aet_ref[...] = lax.dot_general(
        mt, ea_ref[...], (((0,), (1,)), ((), ())),
        preferred_element_type=jnp.float32)


def _k1b(edge_attr, we, att_edge):
    e, ed = edge_attr.shape
    be = 640
    grid = e // be
    return pl.pallas_call(
        _k1b_body,
        grid=(grid,),
        in_specs=[
            pl.BlockSpec((be, ed), lambda b: (b, 0)),
            pl.BlockSpec((ed, HC), lambda b: (0, 0)),
            pl.BlockSpec((H, C), lambda b: (0, 0)),
        ],
        out_specs=pl.BlockSpec((H, be), lambda b: (0, b)),
        out_shape=jax.ShapeDtypeStruct((H, EPAD), jnp.float32),
    )(edge_attr, we, att_edge)


def _k2_body(asrc_hbm, adst_hbm, ei_hbm, aet_hbm,
             expt_hbm, tbl_hbm,
             asrc_v, adst_v, srcb, dstb, aeb, expb, valb,
             idx0, idx1, idx2, idx3, idx4, idx5, idx6, idx7, idx8,
             zerob, tbl_s):
    cid = lax.axis_index("c")
    sid = lax.axis_index("s")
    wid = cid * 16 + sid
    idxs = [idx0, idx1, idx2, idx3, idx4, idx5, idx6, idx7, idx8]

    pltpu.sync_copy(asrc_hbm, asrc_v)
    pltpu.sync_copy(adst_hbm, adst_v)

    z16f = jnp.zeros((16,), jnp.float32)

    def zero_body(i, _):
        zerob[pl.ds(i * 16, 16)] = z16f
        return 0
    lax.fori_loop(0, BATCH // 16, zero_body, 0)
    for i in range(TSTRIPE // BATCH):
        pltpu.sync_copy(zerob, tbl_s.at[pl.ds(sid * TSTRIPE + i * BATCH, BATCH)])
    plsc.subcore_barrier()

    ebase = wid * (EPAD // 32)

    def batch_body(b, _):
        off = ebase + b * BATCH
        pltpu.sync_copy(ei_hbm.at[0, pl.ds(off, BATCH)], srcb)
        pltpu.sync_copy(ei_hbm.at[1, pl.ds(off, BATCH)], dstb)
        for h in range(H):
            pltpu.sync_copy(aet_hbm.at[h, pl.ds(off, BATCH)],
                            aeb.at[pl.ds(h * BATCH, BATCH)])

        def chunk_body(j, _):
            ds16 = pl.ds(j * 16, 16)
            sv = srcb[ds16]
            dv = dstb[ds16]
            m = sv >= 0
            svc = jnp.maximum(sv, 0)
            dvc = jnp.maximum(dv, 0)
            mf = jnp.where(m, 1.0, 0.0).astype(jnp.float32)
            valb[ds16] = mf
            for k in range(TROWS):
                idxs[k][ds16] = dvc * TROWS + k
            for h in range(H):
                g1 = plsc.load_gather(asrc_v, [svc * H + h])
                g2 = plsc.load_gather(adst_v, [dvc * H + h])
                av = aeb[pl.ds(h * BATCH + j * 16, 16)] * mf
                l = g1 + g2 + av
                l = jnp.maximum(l, 0.2 * l)
                ex = jnp.exp(l) * mf
                expb[pl.ds(h * BATCH + j * 16, 16)] = ex
                valb[pl.ds((1 + h) * BATCH + j * 16, 16)] = av
                valb[pl.ds((5 + h) * BATCH + j * 16, 16)] = ex
            return 0
        lax.fori_loop(0, BATCH // 16, chunk_body, 0)

        for k in range(TROWS):
            pltpu.sync_copy(valb.at[pl.ds(k * BATCH, BATCH)],
                            tbl_s.at[idxs[k]], add=True)
        for h in range(H):
            pltpu.sync_copy(expb.at[pl.ds(h * BATCH, BATCH)],
                            expt_hbm.at[h, pl.ds(off, BATCH)])
        return 0
    lax.fori_loop(0, EPAD // 32 // BATCH, batch_body, 0)

    plsc.subcore_barrier()
    pltpu.sync_copy(tbl_s.at[pl.ds(sid * TSTRIPE, TSTRIPE)],
                    tbl_hbm.at[cid, pl.ds(sid * TSTRIPE, TSTRIPE)])


def _k2(asrc, adst, eip, aetp):
    return pl.kernel(
        _k2_body,
        out_type=[
            jax.ShapeDtypeStruct((H, EPAD), jnp.float32),
            jax.ShapeDtypeStruct((2, TPAD), jnp.float32),
        ],
        mesh=_mesh(),
        compiler_params=_sc_params,
        scratch_types=[
            pltpu.VMEM((NNODE * H,), jnp.float32),
            pltpu.VMEM((NNODE * H,), jnp.float32),
            pltpu.VMEM((BATCH,), jnp.int32),
            pltpu.VMEM((BATCH,), jnp.int32),
            pltpu.VMEM((H * BATCH,), jnp.float32),
            pltpu.VMEM((H * BATCH,), jnp.float32),
            pltpu.VMEM((TROWS * BATCH,), jnp.float32),
        ] + [pltpu.VMEM((BATCH,), jnp.int32) for _ in range(TROWS)]
        + [
            pltpu.VMEM((BATCH,), jnp.float32),
            pltpu.VMEM_SHARED((TPAD,), jnp.float32),
        ],
    )(asrc, adst, eip, aetp)


def _k5_body(xs_hbm, ei_hbm, expt_hbm, msg_hbm,
             srcb, dstb, expb, eidc,
             locr0, locr1, locr2, gbuf0, gbuf1, gbuf2, acc_s,
             sg0, sg1, sg2, ss0, ss1, ss2):
    cid = lax.axis_index("c")
    sid = lax.axis_index("s")
    ebase = sid * EPB
    bufs = [gbuf0, gbuf1, gbuf2]
    locrs = [locr0, locr1, locr2]
    gsems = [sg0, sg1, sg2]
    ssems = [ss0, ss1, ss2]

    z16f = jnp.zeros((16,), jnp.float32)
    lane = jnp.arange(16, dtype=jnp.int32)

    def pass_body(p, _):
        lo = cid * (NPASS * PASS_ROWS) + p * PASS_ROWS
        hi = lo + PASS_ROWS

        for i in range(16):
            def zb_body(q, _):
                gbuf0[i, pl.ds(q * 16, 16)] = z16f
                return 0
            lax.fori_loop(0, HC // 16, zb_body, 0)
        for k in range(ROWS_PER_TILE // 16):
            pltpu.sync_copy(
                gbuf0, acc_s.at[pl.ds(sid * ROWS_PER_TILE + k * 16, 16), :])
        plsc.subcore_barrier()

        def seg_body(t, _):
            off = ebase + t * KB
            pltpu.sync_copy(ei_hbm.at[0, pl.ds(off, KB)], srcb)
            pltpu.sync_copy(ei_hbm.at[1, pl.ds(off, KB)], dstb)
            for h in range(H):
                pltpu.sync_copy(expt_hbm.at[h, pl.ds(off, KB)],
                                expb.at[pl.ds(h * KB, KB)])

            def cmp_body(j, cnt):
                ds16 = pl.ds(j * 16, 16)
                sv = srcb[ds16]
                dv = dstb[ds16]
                m = (dv >= lo) & (dv < hi) & (sv >= 0)
                plsc.store_compressed(eidc.at[pl.ds(cnt, 16)],
                                      lane + j * 16, mask=m)
                return cnt + jnp.sum(m.astype(jnp.int32))
            cnt = lax.fori_loop(0, KB // 16, cmp_body, 0)

            z16i = jnp.zeros((16,), jnp.int32)
            eidc[pl.ds(cnt, 16)] = z16i
            eidc[pl.ds(cnt + 16, 16)] = z16i
            ntrip = (cnt + 15) // 16

            def prep(g, b):
                eidv = eidc[pl.ds(g * 16, 16)]
                srcv = jnp.maximum(plsc.load_gather(srcb, [eidv]), 0)
                pltpu.async_copy(xs_hbm.at[srcv], bufs[b], gsems[b])

            def swait(b):
                pltpu.make_async_copy(
                    bufs[b],
                    acc_s.at[plsc.Indices(locrs[b], ignored_value=-1)],
                    ssems[b]).wait()

            @pl.when(cnt > 0)
            def _():
                prep(0, 0)

                @pl.when(ntrip >= 2)
                def _():
                    prep(1, 1)

                def tri_body(i, _):
                    for b in range(3):
                        g = 3 * i + b

                        @pl.when(g < ntrip)
                        def _():
                            eidv = eidc[pl.ds(g * 16, 16)]
                            validv = (g * 16 + lane) < cnt
                            vmf = jnp.where(
                                validv, 1.0, 0.0).astype(jnp.float32)
                            srcv = jnp.maximum(
                                plsc.load_gather(srcb, [eidv]), 0)
                            pltpu.make_async_copy(
                                xs_hbm.at[srcv], bufs[b], gsems[b]).wait()
                            locv = jnp.where(
                                validv,
                                plsc.load_gather(dstb, [eidv]) - lo, -1)
                            locrs[b][pl.ds(0, 16)] = locv
                            buf = bufs[b]
                            for h in range(H):
                                expv = plsc.load_gather(
                                    expb, [eidv + h * KB]) * vmf
                                for e in range(16):
                                    av = z16f + expv[e]
                                    for q in range(C // 16):
                                        sl = pl.ds(h * C + q * 16, 16)
                                        buf[e, sl] = buf[e, sl] * av
                            pltpu.async_copy(
                                buf,
                                acc_s.at[plsc.Indices(locrs[b],
                                                      ignored_value=-1)],
                                ssems[b], add=True)
                            b2 = (b + 2) % 3

                            @pl.when(g >= 1)
                            def _():
                                swait(b2)

                            @pl.when(g + 2 < ntrip)
                            def _():
                                prep(g + 2, b2)
                    return 0
                lax.fori_loop(0, (ntrip + 2) // 3, tri_body, 0)
                for b in range(3):
                    @pl.when((ntrip - 1) % 3 == b)
                    def _():
                        swait(b)
            return 0
        lax.fori_loop(0, EPB // KB, seg_body, 0)

        plsc.subcore_barrier()
        row0 = lo + sid * ROWS_PER_TILE
        pltpu.sync_copy(
            acc_s.at[pl.ds(sid * ROWS_PER_TILE, ROWS_PER_TILE), :],
            msg_hbm.at[pl.ds(row0, ROWS_PER_TILE), :])
        return 0
    lax.fori_loop(0, NPASS, pass_body, 0)


def _k5(xs, eip, expt):
    return pl.kernel(
        _k5_body,
        out_type=jax.ShapeDtypeStruct((MSGPAD, HC), jnp.float32),
        mesh=_mesh(),
        compiler_params=_sc_params,
        scratch_types=[
            pltpu.VMEM((KB,), jnp.int32),
            pltpu.VMEM((KB,), jnp.int32),
            pltpu.VMEM((H * KB,), jnp.float32),
            pltpu.VMEM((KB + 64,), jnp.int32),
            pltpu.VMEM((16,), jnp.int32),
            pltpu.VMEM((16,), jnp.int32),
            pltpu.VMEM((16,), jnp.int32),
            pltpu.VMEM((16, HC), jnp.float32),
            pltpu.VMEM((16, HC), jnp.float32),
            pltpu.VMEM((16, HC), jnp.float32),
            pltpu.VMEM_SHARED((PASS_ROWS, HC), jnp.float32),
            pltpu.SemaphoreType.DMA,
            pltpu.SemaphoreType.DMA,
            pltpu.SemaphoreType.DMA,
            pltpu.SemaphoreType.DMA,
            pltpu.SemaphoreType.DMA,
            pltpu.SemaphoreType.DMA,
        ],
    )(xs, eip, expt)


def _k6_body(msg_ref, xs_ref, asrc_ref, adst_ref, ta_ref, tb_ref,
             bias_ref, w2_ref, b2_ref, out_ref):
    deg = ta_ref[:, 0:1] + tb_ref[:, 0:1]
    degc = jnp.maximum(deg, 1.0)
    gs = []
    for h in range(H):
        aes = ta_ref[:, 1 + h:2 + h] + tb_ref[:, 1 + h:2 + h]
        dnp = ta_ref[:, 5 + h:6 + h] + tb_ref[:, 5 + h:6 + h]
        ael = aes / degc
        ll = asrc_ref[:, h:h + 1] + adst_ref[:, h:h + 1] + ael
        ll = jnp.maximum(ll, 0.2 * ll)
        exl = jnp.exp(ll)
        den = dnp + exl + 1e-16
        gh = (msg_ref[:, h * C:(h + 1) * C]
              + exl * xs_ref[:, h * C:(h + 1) * C]) / den
        gs.append(gh)
    gat = jnp.concatenate(gs, axis=1) + bias_ref[...]
    out_ref[...] = (jnp.dot(gat, w2_ref[...], preferred_element_type=jnp.float32)
                    + b2_ref[...])


def _k6(msg, xs, asrc, adst, ta, tb, bias, w2, b2):
    n = xs.shape[0]
    f = w2.shape[1]
    bn = 1024
    grid = (n + bn - 1) // bn
    return pl.pallas_call(
        _k6_body,
        grid=(grid,),
        in_specs=[
            pl.BlockSpec((bn, HC), lambda b: (b, 0)),
            pl.BlockSpec((bn, HC), lambda b: (b, 0)),
            pl.BlockSpec((bn, H), lambda b: (b, 0)),
            pl.BlockSpec((bn, H), lambda b: (b, 0)),
            pl.BlockSpec((bn, TROWS), lambda b: (b, 0)),
            pl.BlockSpec((bn, TROWS), lambda b: (b, 0)),
            pl.BlockSpec((1, HC), lambda b: (0, 0)),
            pl.BlockSpec((HC, f), lambda b: (0, 0)),
            pl.BlockSpec((1, f), lambda b: (0, 0)),
        ],
        out_specs=pl.BlockSpec((bn, f), lambda b: (b, 0)),
        out_shape=jax.ShapeDtypeStruct((n, f), jnp.float32),
    )(msg, xs, asrc, adst, ta, tb, bias, w2, b2)


def kernel(x, edge_index, edge_attr, W, att_src, att_dst, att_edge,
           We, bias_gat, W2, b2):
    npad = EPAD - edge_index.shape[1]
    eip = jnp.concatenate(
        [edge_index.astype(jnp.int32),
         jnp.full((2, npad), -1, jnp.int32)], axis=1)

    xs, asrc, adst = _k1(x, W, att_src, att_dst)
    aetp = _k1b(edge_attr, We, att_edge)

    expt, tbl = _k2(asrc.reshape(-1), adst.reshape(-1), eip, aetp)
    msg = _k5(xs, eip, expt)

    ta = tbl[0, :TFLAT].reshape(NNODE, TROWS)
    tb = tbl[1, :TFLAT].reshape(NNODE, TROWS)
    return _k6(msg, xs, asrc, adst, ta, tb,
               bias_gat.reshape(1, HC), W2, b2.reshape(1, -1))
